# baseline (speedup 1.0000x reference)
import functools

import jax
import jax.numpy as jnp
from jax import lax
from jax.experimental import pallas as pl
from jax.experimental.pallas import tpu as pltpu
from jax.experimental.pallas import tpu_sc as plsc

N = 10000
E = 320000
D = 128
NLAYERS = 4
NC, NS = 2, 16
NW = NC * NS
CHUNK = 128
NCH = 81
EW = NCH * CHUNK
E_PAD = NW * EW
E_HALF = E_PAD // 2
NCHT = E_HALF // (NS * CHUNK)
ACC_ROWS = 10112

f32 = jnp.float32
bf16 = jnp.bfloat16
i32 = jnp.int32


def _core_gather(sid, tbl_hbm, tbl_sh, idx_hbm, out_hbm, bufs, ibs,
                 sgs, sws, sis):
    @pl.when(sid < NS - 1)
    def _load_full():
        pltpu.sync_copy(tbl_hbm.at[pl.ds(sid * 640, 640), :],
                        tbl_sh.at[pl.ds(sid * 640, 640), :])

    @pl.when(sid == NS - 1)
    def _load_tail():
        pltpu.sync_copy(tbl_hbm.at[pl.ds((NS - 1) * 640, N - (NS - 1) * 640), :],
                        tbl_sh.at[pl.ds((NS - 1) * 640, N - (NS - 1) * 640), :])

    plsc.subcore_barrier()
    base = sid * NCHT

    def fire_idx(b, j):
        pltpu.async_copy(idx_hbm.at[sid, pl.ds(j, 1), :], ibs[b], sis[b])

    def wait_idx(b):
        pltpu.make_async_copy(
            idx_hbm.at[sid, pl.ds(0, 1), :], ibs[b], sis[b]).wait()

    def fire_gather(b):
        pltpu.async_copy(tbl_sh.at[ibs[b].at[0]], bufs[b], sgs[b])

    def wait_gather(b):
        pltpu.make_async_copy(
            tbl_sh.at[ibs[b].at[0]], bufs[b], sgs[b]).wait()

    def wait_wb(b):
        pltpu.make_async_copy(
            bufs[b], out_hbm.at[pl.ds(0, CHUNK), :], sws[b]).wait()

    for b in range(3):
        fire_idx(b, b)
    for b in range(2):
        wait_idx(b)
        fire_gather(b)

    def group(g, carry):
        for b in range(3):
            j = g * 3 + b
            b2 = (b + 2) % 3
            wait_gather(b)
            pltpu.async_copy(
                bufs[b], out_hbm.at[pl.ds((base + j) * CHUNK, CHUNK), :],
                sws[b])

            @pl.when(j >= 1)
            def _wait_prev_wb():
                wait_wb(b2)

            @pl.when(j + 2 < NCHT)
            def _fire_next_gather():
                wait_idx(b2)
                fire_gather(b2)

            @pl.when(j + 3 < NCHT)
            def _fire_next_idx():
                fire_idx(b, j + 3)
        return carry

    lax.fori_loop(0, NCHT // 3, group, 0)
    wait_wb((NCHT - 1) % 3)


def _gather_body(p_hbm, q_hbm, rowg_hbm, colg_hbm, tp_hbm, tq_hbm,
                 b0, b1, b2, ib0, ib1, ib2, tbl_sh,
                 sg0, sg1, sg2, sw0, sw1, sw2, si0, si1, si2):
    cid = lax.axis_index("c")
    sid = lax.axis_index("s")
    bufs = (b0, b1, b2)
    ibs = (ib0, ib1, ib2)
    sgs = (sg0, sg1, sg2)
    sws = (sw0, sw1, sw2)
    sis = (si0, si1, si2)

    @pl.when(cid == 0)
    def _core0():
        _core_gather(sid, p_hbm, tbl_sh, rowg_hbm, tp_hbm, bufs, ibs,
                     sgs, sws, sis)

    @pl.when(cid == 1)
    def _core1():
        _core_gather(sid, q_hbm, tbl_sh, colg_hbm, tq_hbm, bufs, ibs,
                     sgs, sws, sis)


@functools.cache
def _get_gather():
    return pl.kernel(
        _gather_body,
        out_type=[jax.ShapeDtypeStruct((E_HALF, D), f32),
                  jax.ShapeDtypeStruct((E_HALF, D), f32)],
        mesh=plsc.VectorSubcoreMesh(core_axis_name="c", subcore_axis_name="s"),
        scratch_types=(
            [pltpu.VMEM((CHUNK, D), f32)] * 3
            + [pltpu.VMEM((1, CHUNK), i32)] * 3
            + [pltpu.VMEM_SHARED((N, D), f32)]
            + [pltpu.SemaphoreType.DMA] * 9
        ),
    )


def _scatter_body(m0_hbm, m1_hbm, rows_hbm, out_hbm, mb0, mb1, mb2,
                  ib0, ib1, ib2, acc, sm0, sm1, sm2, si0, si1, si2,
                  sz, sa0, sa1, sa2):
    cid = lax.axis_index("c")
    sid = lax.axis_index("s")
    wid = sid * NC + cid
    mbs = (mb0, mb1, mb2)
    ibs = (ib0, ib1, ib2)
    sms = (sm0, sm1, sm2)
    sis = (si0, si1, si2)
    sas = (sa0, sa1, sa2)
    base = wid * NCH

    def fire(b, j):
        @pl.when(wid < NW // 2)
        def _fire0():
            pltpu.async_copy(
                m0_hbm.at[pl.ds((base + j) * CHUNK, CHUNK), :], mbs[b], sms[b])

        @pl.when(wid >= NW // 2)
        def _fire1():
            pltpu.async_copy(
                m1_hbm.at[pl.ds((base - NW // 2 * NCH + j) * CHUNK, CHUNK), :],
                mbs[b], sms[b])

        pltpu.async_copy(rows_hbm.at[wid, pl.ds(j, 1), :], ibs[b], sis[b])

    def zrow(r, carry):
        for c8 in range(D // 16):
            mb0[r, pl.ds(c8 * 16, 16)] = jnp.zeros((16,), f32)
        return carry

    lax.fori_loop(0, CHUNK, zrow, 0)
    tbase = sid * (ACC_ROWS // NS)
    for k in range(4):
        pltpu.async_copy(mb0, acc.at[pl.ds(tbase + k * CHUNK, CHUNK), :], sz)
    pltpu.async_copy(mb0.at[pl.ds(0, 120), :],
                     acc.at[pl.ds(tbase + 4 * CHUNK, 120), :], sz)
    fire(1, 1)
    fire(2, 2)
    for k in range(4):
        pltpu.make_async_copy(
            mb0, acc.at[pl.ds(tbase, CHUNK), :], sz).wait()
    pltpu.make_async_copy(
        mb0.at[pl.ds(0, 120), :], acc.at[pl.ds(tbase, 120), :], sz).wait()
    fire(0, 0)
    plsc.subcore_barrier()

    def group(g, carry):
        for b in range(3):
            j = g * 3 + b
            pltpu.make_async_copy(
                m0_hbm.at[pl.ds(0, CHUNK), :], mbs[b], sms[b]).wait()
            pltpu.make_async_copy(
                rows_hbm.at[wid, pl.ds(0, 1), :], ibs[b], sis[b]).wait()
            pltpu.sync_copy(mbs[b], acc.at[ibs[b].at[0]], add=True)

            @pl.when(j + 3 < NCH)
            def _refill():
                fire(b, j + 3)
        return carry

    lax.fori_loop(0, NCH // 3, group, 0)
    plsc.subcore_barrier()

    @pl.when(sid < NS - 1)
    def _copy_full():
        pltpu.sync_copy(acc.at[pl.ds(sid * 640, 640), :],
                        out_hbm.at[cid, pl.ds(sid * 640, 640), :])

    @pl.when(sid == NS - 1)
    def _copy_tail():
        pltpu.sync_copy(acc.at[pl.ds((NS - 1) * 640, N - (NS - 1) * 640), :],
                        out_hbm.at[cid, pl.ds((NS - 1) * 640, N - (NS - 1) * 640), :])


@functools.cache
def _get_scatter():
    return pl.kernel(
        _scatter_body,
        out_type=jax.ShapeDtypeStruct((NC, N, D), f32),
        mesh=plsc.VectorSubcoreMesh(core_axis_name="c", subcore_axis_name="s"),
        scratch_types=[
            pltpu.VMEM((CHUNK, D), f32), pltpu.VMEM((CHUNK, D), f32),
            pltpu.VMEM((CHUNK, D), f32),
            pltpu.VMEM((1, CHUNK), i32), pltpu.VMEM((1, CHUNK), i32),
            pltpu.VMEM((1, CHUNK), i32),
            pltpu.VMEM_SHARED((ACC_ROWS, D), f32),
            pltpu.SemaphoreType.DMA, pltpu.SemaphoreType.DMA,
            pltpu.SemaphoreType.DMA, pltpu.SemaphoreType.DMA,
            pltpu.SemaphoreType.DMA, pltpu.SemaphoreType.DMA,
            pltpu.SemaphoreType.DMA, pltpu.SemaphoreType.DMA,
            pltpu.SemaphoreType.DMA, pltpu.SemaphoreType.DMA,
        ],
    )


def _embed_tc(x_ref, ew_ref, eb_ref, w1a_ref, b1_ref, w1b_ref,
              h_ref, p_ref, q_ref):
    h = jnp.dot(x_ref[...], ew_ref[...], preferred_element_type=f32) + eb_ref[...]
    h_ref[...] = h
    hb = h.astype(bf16)
    p_ref[...] = jnp.dot(hb, w1a_ref[...].astype(bf16),
                         preferred_element_type=f32) + b1_ref[...]
    q_ref[...] = jnp.dot(hb, w1b_ref[...].astype(bf16),
                         preferred_element_type=f32)


BN = 2000
_full = lambda a, b: pl.BlockSpec((a, b), lambda j: (0, 0))
_blk = lambda: pl.BlockSpec((BN, D), lambda j: (j, 0))

_embed = pl.pallas_call(
    _embed_tc,
    grid=(N // BN,),
    in_specs=[_blk(), _full(D, D), _full(1, D), _full(D, D), _full(1, D),
              _full(D, D)],
    out_specs=[_blk(), _blk(), _blk()],
    out_shape=[jax.ShapeDtypeStruct((N, D), f32)] * 3,
)


def _edge_tc(tp_ref, tq_ref, attr_ref, w1c_ref, w2_ref, b2_ref, m_ref):
    u = tp_ref[...] + tq_ref[...] + attr_ref[...] * w1c_ref[...]
    u = u * jax.nn.sigmoid(u)
    v = jnp.dot(u.astype(bf16), w2_ref[...],
                preferred_element_type=f32) + b2_ref[...]
    m_ref[...] = v * jax.nn.sigmoid(v)


BE = 1024
_edge_mlp = pl.pallas_call(
    _edge_tc,
    grid=(E_HALF // BE,),
    in_specs=[pl.BlockSpec((BE, D), lambda j: (j, 0)),
              pl.BlockSpec((BE, D), lambda j: (j, 0)),
              pl.BlockSpec((BE, 1), lambda j: (j, 0)),
              _full(1, D), _full(D, D), _full(1, D)],
    out_specs=pl.BlockSpec((BE, D), lambda j: (j, 0)),
    out_shape=jax.ShapeDtypeStruct((E_HALF, D), f32),
)


def _node_tc(h_ref, a0_ref, a1_ref, nw1a_ref, nw1b_ref, nb1_ref, nw2_ref,
             nb2_ref, w1a_ref, b1_ref, w1b_ref, h_out, p_out, q_out):
    h = h_ref[...]
    agg = a0_ref[...] + a1_ref[...]
    t = (jnp.dot(h, nw1a_ref[...], preferred_element_type=f32)
         + jnp.dot(agg, nw1b_ref[...], preferred_element_type=f32)
         + nb1_ref[...])
    t = t * jax.nn.sigmoid(t)
    hn = jnp.dot(t, nw2_ref[...], preferred_element_type=f32) + nb2_ref[...]
    h_out[...] = hn
    hb = hn.astype(bf16)
    p_out[...] = jnp.dot(hb, w1a_ref[...].astype(bf16),
                         preferred_element_type=f32) + b1_ref[...]
    q_out[...] = jnp.dot(hb, w1b_ref[...].astype(bf16),
                         preferred_element_type=f32)


_node = pl.pallas_call(
    _node_tc,
    grid=(N // BN,),
    in_specs=[_blk(), _blk(), _blk(),
              _full(D, D), _full(D, D), _full(1, D), _full(D, D), _full(1, D),
              _full(D, D), _full(1, D), _full(D, D)],
    out_specs=[_blk(), _blk(), _blk()],
    out_shape=[jax.ShapeDtypeStruct((N, D), f32)] * 3,
)


def _node_dec_tc(h_ref, a0_ref, a1_ref, nw1a_ref, nw1b_ref, nb1_ref,
                 nw2_ref, nb2_ref, dw1_ref, db1_ref, dw2_ref, db2_ref, o_ref):
    h = h_ref[...]
    agg = a0_ref[...] + a1_ref[...]
    t = (jnp.dot(h, nw1a_ref[...], preferred_element_type=f32)
         + jnp.dot(agg, nw1b_ref[...], preferred_element_type=f32)
         + nb1_ref[...])
    t = t * jax.nn.sigmoid(t)
    hn = jnp.dot(t, nw2_ref[...], preferred_element_type=f32) + nb2_ref[...]
    t2 = jnp.dot(hn, dw1_ref[...], preferred_element_type=f32) + db1_ref[...]
    t2 = t2 * jax.nn.sigmoid(t2)
    o_ref[...] = jnp.dot(t2, dw2_ref[...], preferred_element_type=f32) + db2_ref[...]


_node_dec = pl.pallas_call(
    _node_dec_tc,
    grid=(N // BN,),
    in_specs=[_blk(), _blk(), _blk(),
              _full(D, D), _full(D, D), _full(1, D), _full(D, D), _full(1, D),
              _full(D, D), _full(1, D), _full(D, 3), _full(1, 3)],
    out_specs=pl.BlockSpec((BN, 3), lambda j: (j, 0)),
    out_shape=jax.ShapeDtypeStruct((N, 3), f32),
)


def kernel(nodes, edges, edge_attr, emb_W, emb_b, edge_W1, edge_b1, edge_W2,
           edge_b2, node_W1, node_b1, node_W2, node_b2, dec_W1, dec_b1,
           dec_W2, dec_b2):
    row = edges[0]
    col = edges[1]
    padz = jnp.zeros((E_PAD - E,), i32)
    padn = N + jnp.arange(E_PAD - E, dtype=i32) % (ACC_ROWS - N)
    rowg = jnp.concatenate([row, padz]).reshape(2, NS, NCHT, CHUNK)
    colg = jnp.concatenate([col, padz]).reshape(2, NS, NCHT, CHUNK)
    rowsc = jnp.concatenate([row, padn]).reshape(NW, NCH, CHUNK)
    attr_p = jnp.concatenate(
        [edge_attr, jnp.zeros((E_PAD - E, 1), f32)], axis=0)
    attr_h = (attr_p[:E_HALF], attr_p[E_HALF:])

    h, p, q = _embed(nodes, emb_W, emb_b.reshape(1, D),
                     edge_W1[0, :D], edge_b1[0].reshape(1, D),
                     edge_W1[0, D:2 * D])
    out = None
    gather_fn = _get_gather()
    scatter_fn = _get_scatter()
    for i in range(NLAYERS):
        ms = []
        for half in range(2):
            tp, tq = gather_fn(p, q, rowg[half], colg[half])
            ms.append(_edge_mlp(tp, tq, attr_h[half],
                                edge_W1[i, 2 * D:].reshape(1, D),
                                edge_W2[i].astype(bf16),
                                edge_b2[i].reshape(1, D)))
        agg2 = scatter_fn(ms[0], ms[1], rowsc)
        nw1a = node_W1[i, :D]
        nw1b = node_W1[i, D:]
        if i + 1 < NLAYERS:
            h, p, q = _node(h, agg2[0], agg2[1], nw1a, nw1b,
                            node_b1[i].reshape(1, D), node_W2[i],
                            node_b2[i].reshape(1, D),
                            edge_W1[i + 1, :D], edge_b1[i + 1].reshape(1, D),
                            edge_W1[i + 1, D:2 * D])
        else:
            out = _node_dec(h, agg2[0], agg2[1], nw1a, nw1b,
                            node_b1[i].reshape(1, D), node_W2[i],
                            node_b2[i].reshape(1, D),
                            dec_W1, dec_b1.reshape(1, D),
                            dec_W2, dec_b2.reshape(1, 3))
    return out


# speedup vs baseline: 1.0010x; 1.0010x over previous
import functools

import jax
import jax.numpy as jnp
from jax import lax
from jax.experimental import pallas as pl
from jax.experimental.pallas import tpu as pltpu
from jax.experimental.pallas import tpu_sc as plsc

N = 10000
E = 320000
D = 128
NLAYERS = 4
NC, NS = 2, 16
NW = NC * NS
CHUNK = 128
NCH = 81
EW = NCH * CHUNK
E_PAD = NW * EW
E_HALF = E_PAD // 2
NCHT = E_HALF // (NS * CHUNK)
ACC_ROWS = 10112

f32 = jnp.float32
bf16 = jnp.bfloat16
i32 = jnp.int32


def _core_gather(sid, tbl_hbm, tbl_sh, idx_hbm, out_hbm, bufs, ibs,
                 sgs, sws, sis):
    @pl.when(sid < NS - 1)
    def _load_full():
        pltpu.sync_copy(tbl_hbm.at[pl.ds(sid * 640, 640), :],
                        tbl_sh.at[pl.ds(sid * 640, 640), :])

    @pl.when(sid == NS - 1)
    def _load_tail():
        pltpu.sync_copy(tbl_hbm.at[pl.ds((NS - 1) * 640, N - (NS - 1) * 640), :],
                        tbl_sh.at[pl.ds((NS - 1) * 640, N - (NS - 1) * 640), :])

    plsc.subcore_barrier()
    base = sid * NCHT

    def fire_idx(b, j):
        pltpu.async_copy(idx_hbm.at[sid, pl.ds(j, 1), :], ibs[b], sis[b])

    def wait_idx(b):
        pltpu.make_async_copy(
            idx_hbm.at[sid, pl.ds(0, 1), :], ibs[b], sis[b]).wait()

    def fire_gather(b):
        pltpu.async_copy(tbl_sh.at[ibs[b].at[0]], bufs[b], sgs[b])

    def wait_gather(b):
        pltpu.make_async_copy(
            tbl_sh.at[ibs[b].at[0]], bufs[b], sgs[b]).wait()

    def wait_wb(b):
        pltpu.make_async_copy(
            bufs[b], out_hbm.at[pl.ds(0, CHUNK), :], sws[b]).wait()

    for b in range(3):
        fire_idx(b, b)
    for b in range(2):
        wait_idx(b)
        fire_gather(b)

    def group(g, carry):
        for b in range(3):
            j = g * 3 + b
            b2 = (b + 2) % 3
            wait_gather(b)
            pltpu.async_copy(
                bufs[b], out_hbm.at[pl.ds((base + j) * CHUNK, CHUNK), :],
                sws[b])

            @pl.when(j >= 1)
            def _wait_prev_wb():
                wait_wb(b2)

            @pl.when(j + 2 < NCHT)
            def _fire_next_gather():
                wait_idx(b2)
                fire_gather(b2)

            @pl.when(j + 3 < NCHT)
            def _fire_next_idx():
                fire_idx(b, j + 3)
        return carry

    lax.fori_loop(0, NCHT // 3, group, 0)
    wait_wb((NCHT - 1) % 3)


def _gather_body(p_hbm, q_hbm, rowg_hbm, colg_hbm, tp_hbm, tq_hbm,
                 b0, b1, b2, ib0, ib1, ib2, tbl_sh,
                 sg0, sg1, sg2, sw0, sw1, sw2, si0, si1, si2):
    cid = lax.axis_index("c")
    sid = lax.axis_index("s")
    bufs = (b0, b1, b2)
    ibs = (ib0, ib1, ib2)
    sgs = (sg0, sg1, sg2)
    sws = (sw0, sw1, sw2)
    sis = (si0, si1, si2)

    @pl.when(cid == 0)
    def _core0():
        _core_gather(sid, p_hbm, tbl_sh, rowg_hbm, tp_hbm, bufs, ibs,
                     sgs, sws, sis)

    @pl.when(cid == 1)
    def _core1():
        _core_gather(sid, q_hbm, tbl_sh, colg_hbm, tq_hbm, bufs, ibs,
                     sgs, sws, sis)


@functools.cache
def _get_gather():
    return pl.kernel(
        _gather_body,
        out_type=[jax.ShapeDtypeStruct((E_HALF, D), f32),
                  jax.ShapeDtypeStruct((E_HALF, D), f32)],
        mesh=plsc.VectorSubcoreMesh(core_axis_name="c", subcore_axis_name="s"),
        scratch_types=(
            [pltpu.VMEM((CHUNK, D), f32)] * 3
            + [pltpu.VMEM((1, CHUNK), i32)] * 3
            + [pltpu.VMEM_SHARED((N, D), f32)]
            + [pltpu.SemaphoreType.DMA] * 9
        ),
    )


def _scatter_body(m0_hbm, m1_hbm, rows_hbm, out_hbm, mb0, mb1, mb2,
                  ib0, ib1, ib2, acc, sm0, sm1, sm2, si0, si1, si2,
                  sz, sa0, sa1, sa2):
    cid = lax.axis_index("c")
    sid = lax.axis_index("s")
    wid = sid * NC + cid
    mbs = (mb0, mb1, mb2)
    ibs = (ib0, ib1, ib2)
    sms = (sm0, sm1, sm2)
    sis = (si0, si1, si2)
    sas = (sa0, sa1, sa2)
    base = wid * NCH

    def fire(b, j):
        @pl.when(wid < NW // 2)
        def _fire0():
            pltpu.async_copy(
                m0_hbm.at[pl.ds((base + j) * CHUNK, CHUNK), :], mbs[b], sms[b])

        @pl.when(wid >= NW // 2)
        def _fire1():
            pltpu.async_copy(
                m1_hbm.at[pl.ds((base - NW // 2 * NCH + j) * CHUNK, CHUNK), :],
                mbs[b], sms[b])

        pltpu.async_copy(rows_hbm.at[wid, pl.ds(j, 1), :], ibs[b], sis[b])

    def zrow(r, carry):
        for c8 in range(D // 16):
            mb0[r, pl.ds(c8 * 16, 16)] = jnp.zeros((16,), f32)
        return carry

    lax.fori_loop(0, CHUNK, zrow, 0)
    tbase = sid * (ACC_ROWS // NS)
    for k in range(4):
        pltpu.async_copy(mb0, acc.at[pl.ds(tbase + k * CHUNK, CHUNK), :], sz)
    pltpu.async_copy(mb0.at[pl.ds(0, 120), :],
                     acc.at[pl.ds(tbase + 4 * CHUNK, 120), :], sz)
    fire(1, 1)
    fire(2, 2)
    for k in range(4):
        pltpu.make_async_copy(
            mb0, acc.at[pl.ds(tbase, CHUNK), :], sz).wait()
    pltpu.make_async_copy(
        mb0.at[pl.ds(0, 120), :], acc.at[pl.ds(tbase, 120), :], sz).wait()
    fire(0, 0)
    plsc.subcore_barrier()

    def group(g, carry):
        for b in range(3):
            j = g * 3 + b
            pltpu.make_async_copy(
                m0_hbm.at[pl.ds(0, CHUNK), :], mbs[b], sms[b]).wait()
            pltpu.make_async_copy(
                rows_hbm.at[wid, pl.ds(0, 1), :], ibs[b], sis[b]).wait()
            pltpu.sync_copy(mbs[b], acc.at[ibs[b].at[0]], add=True)

            @pl.when(j + 3 < NCH)
            def _refill():
                fire(b, j + 3)
        return carry

    lax.fori_loop(0, NCH // 3, group, 0)
    plsc.subcore_barrier()

    @pl.when(sid < NS - 1)
    def _copy_full():
        pltpu.sync_copy(acc.at[pl.ds(sid * 640, 640), :],
                        out_hbm.at[cid, pl.ds(sid * 640, 640), :])

    @pl.when(sid == NS - 1)
    def _copy_tail():
        pltpu.sync_copy(acc.at[pl.ds((NS - 1) * 640, N - (NS - 1) * 640), :],
                        out_hbm.at[cid, pl.ds((NS - 1) * 640, N - (NS - 1) * 640), :])


@functools.cache
def _get_scatter():
    return pl.kernel(
        _scatter_body,
        out_type=jax.ShapeDtypeStruct((NC, N, D), f32),
        mesh=plsc.VectorSubcoreMesh(core_axis_name="c", subcore_axis_name="s"),
        scratch_types=[
            pltpu.VMEM((CHUNK, D), f32), pltpu.VMEM((CHUNK, D), f32),
            pltpu.VMEM((CHUNK, D), f32),
            pltpu.VMEM((1, CHUNK), i32), pltpu.VMEM((1, CHUNK), i32),
            pltpu.VMEM((1, CHUNK), i32),
            pltpu.VMEM_SHARED((ACC_ROWS, D), f32),
            pltpu.SemaphoreType.DMA, pltpu.SemaphoreType.DMA,
            pltpu.SemaphoreType.DMA, pltpu.SemaphoreType.DMA,
            pltpu.SemaphoreType.DMA, pltpu.SemaphoreType.DMA,
            pltpu.SemaphoreType.DMA, pltpu.SemaphoreType.DMA,
            pltpu.SemaphoreType.DMA, pltpu.SemaphoreType.DMA,
        ],
    )


def _embed_tc(x_ref, ew_ref, eb_ref, w1a_ref, b1_ref, w1b_ref,
              h_ref, p_ref, q_ref):
    h = jnp.dot(x_ref[...], ew_ref[...], preferred_element_type=f32) + eb_ref[...]
    h_ref[...] = h
    p_ref[...] = jnp.dot(h, w1a_ref[...], preferred_element_type=f32) + b1_ref[...]
    q_ref[...] = jnp.dot(h, w1b_ref[...], preferred_element_type=f32)


BN = 2000
_full = lambda a, b: pl.BlockSpec((a, b), lambda j: (0, 0))
_blk = lambda: pl.BlockSpec((BN, D), lambda j: (j, 0))

_embed = pl.pallas_call(
    _embed_tc,
    grid=(N // BN,),
    in_specs=[_blk(), _full(D, D), _full(1, D), _full(D, D), _full(1, D),
              _full(D, D)],
    out_specs=[_blk(), _blk(), _blk()],
    out_shape=[jax.ShapeDtypeStruct((N, D), f32)] * 3,
)


def _edge_tc(tp_ref, tq_ref, attr_ref, w1c_ref, w2_ref, b2_ref, m_ref):
    u = tp_ref[...] + tq_ref[...] + attr_ref[...] * w1c_ref[...]
    u = u * jax.nn.sigmoid(u)
    v = jnp.dot(u.astype(bf16), w2_ref[...],
                preferred_element_type=f32) + b2_ref[...]
    m_ref[...] = v * jax.nn.sigmoid(v)


BE = 1024
_edge_mlp = pl.pallas_call(
    _edge_tc,
    grid=(E_HALF // BE,),
    in_specs=[pl.BlockSpec((BE, D), lambda j: (j, 0)),
              pl.BlockSpec((BE, D), lambda j: (j, 0)),
              pl.BlockSpec((BE, 1), lambda j: (j, 0)),
              _full(1, D), _full(D, D), _full(1, D)],
    out_specs=pl.BlockSpec((BE, D), lambda j: (j, 0)),
    out_shape=jax.ShapeDtypeStruct((E_HALF, D), f32),
)


def _node_tc(h_ref, a0_ref, a1_ref, nw1a_ref, nw1b_ref, nb1_ref, nw2_ref,
             nb2_ref, w1a_ref, b1_ref, w1b_ref, h_out, p_out, q_out):
    h = h_ref[...]
    agg = a0_ref[...] + a1_ref[...]
    t = (jnp.dot(h, nw1a_ref[...], preferred_element_type=f32)
         + jnp.dot(agg, nw1b_ref[...], preferred_element_type=f32)
         + nb1_ref[...])
    t = t * jax.nn.sigmoid(t)
    hn = jnp.dot(t, nw2_ref[...], preferred_element_type=f32) + nb2_ref[...]
    h_out[...] = hn
    p_out[...] = jnp.dot(hn, w1a_ref[...], preferred_element_type=f32) + b1_ref[...]
    q_out[...] = jnp.dot(hn, w1b_ref[...], preferred_element_type=f32)


_node = pl.pallas_call(
    _node_tc,
    grid=(N // BN,),
    in_specs=[_blk(), _blk(), _blk(),
              _full(D, D), _full(D, D), _full(1, D), _full(D, D), _full(1, D),
              _full(D, D), _full(1, D), _full(D, D)],
    out_specs=[_blk(), _blk(), _blk()],
    out_shape=[jax.ShapeDtypeStruct((N, D), f32)] * 3,
)


def _node_dec_tc(h_ref, a0_ref, a1_ref, nw1a_ref, nw1b_ref, nb1_ref,
                 nw2_ref, nb2_ref, dw1_ref, db1_ref, dw2_ref, db2_ref, o_ref):
    h = h_ref[...]
    agg = a0_ref[...] + a1_ref[...]
    t = (jnp.dot(h, nw1a_ref[...], preferred_element_type=f32)
         + jnp.dot(agg, nw1b_ref[...], preferred_element_type=f32)
         + nb1_ref[...])
    t = t * jax.nn.sigmoid(t)
    hn = jnp.dot(t, nw2_ref[...], preferred_element_type=f32) + nb2_ref[...]
    t2 = jnp.dot(hn, dw1_ref[...], preferred_element_type=f32) + db1_ref[...]
    t2 = t2 * jax.nn.sigmoid(t2)
    o_ref[...] = jnp.dot(t2, dw2_ref[...], preferred_element_type=f32) + db2_ref[...]


_node_dec = pl.pallas_call(
    _node_dec_tc,
    grid=(N // BN,),
    in_specs=[_blk(), _blk(), _blk(),
              _full(D, D), _full(D, D), _full(1, D), _full(D, D), _full(1, D),
              _full(D, D), _full(1, D), _full(D, 3), _full(1, 3)],
    out_specs=pl.BlockSpec((BN, 3), lambda j: (j, 0)),
    out_shape=jax.ShapeDtypeStruct((N, 3), f32),
)


def kernel(nodes, edges, edge_attr, emb_W, emb_b, edge_W1, edge_b1, edge_W2,
           edge_b2, node_W1, node_b1, node_W2, node_b2, dec_W1, dec_b1,
           dec_W2, dec_b2):
    row = edges[0]
    col = edges[1]
    padz = jnp.zeros((E_PAD - E,), i32)
    padn = N + jnp.arange(E_PAD - E, dtype=i32) % (ACC_ROWS - N)
    rowg = jnp.concatenate([row, padz]).reshape(2, NS, NCHT, CHUNK)
    colg = jnp.concatenate([col, padz]).reshape(2, NS, NCHT, CHUNK)
    rowsc = jnp.concatenate([row, padn]).reshape(NW, NCH, CHUNK)
    attr_p = jnp.concatenate(
        [edge_attr, jnp.zeros((E_PAD - E, 1), f32)], axis=0)
    attr_h = (attr_p[:E_HALF], attr_p[E_HALF:])

    h, p, q = _embed(nodes, emb_W, emb_b.reshape(1, D),
                     edge_W1[0, :D], edge_b1[0].reshape(1, D),
                     edge_W1[0, D:2 * D])
    out = None
    gather_fn = _get_gather()
    scatter_fn = _get_scatter()
    for i in range(NLAYERS):
        ms = []
        for half in range(2):
            tp, tq = gather_fn(p, q, rowg[half], colg[half])
            ms.append(_edge_mlp(tp, tq, attr_h[half],
                                edge_W1[i, 2 * D:].reshape(1, D),
                                edge_W2[i].astype(bf16),
                                edge_b2[i].reshape(1, D)))
        agg2 = scatter_fn(ms[0], ms[1], rowsc)
        nw1a = node_W1[i, :D]
        nw1b = node_W1[i, D:]
        if i + 1 < NLAYERS:
            h, p, q = _node(h, agg2[0], agg2[1], nw1a, nw1b,
                            node_b1[i].reshape(1, D), node_W2[i],
                            node_b2[i].reshape(1, D),
                            edge_W1[i + 1, :D], edge_b1[i + 1].reshape(1, D),
                            edge_W1[i + 1, D:2 * D])
        else:
            out = _node_dec(h, agg2[0], agg2[1], nw1a, nw1b,
                            node_b1[i].reshape(1, D), node_W2[i],
                            node_b2[i].reshape(1, D),
                            dec_W1, dec_b1.reshape(1, D),
                            dec_W2, dec_b2.reshape(1, 3))
    return out


# speedup vs baseline: 1.0015x; 1.0006x over previous
import functools

import jax
import jax.numpy as jnp
from jax import lax
from jax.experimental import pallas as pl
from jax.experimental.pallas import tpu as pltpu
from jax.experimental.pallas import tpu_sc as plsc

N = 10000
E = 320000
D = 128
NLAYERS = 4
NC, NS = 2, 16
NW = NC * NS
CHUNK = 128
NCH = 81
EW = NCH * CHUNK
E_PAD = NW * EW
E_HALF = E_PAD // 2
NCHT = E_HALF // (NS * CHUNK)
ACC_ROWS = 10112

f32 = jnp.float32
bf16 = jnp.bfloat16
i32 = jnp.int32


def _core_gather(sid, tbl_hbm, tbl_sh, idx_hbm, out_hbm, bufs, ibs,
                 sgs, sws, sis):
    base = sid * NCHT

    def fire_idx(b, j):
        pltpu.async_copy(idx_hbm.at[sid, pl.ds(j, 1), :], ibs[b], sis[b])

    def wait_idx(b):
        pltpu.make_async_copy(
            idx_hbm.at[sid, pl.ds(0, 1), :], ibs[b], sis[b]).wait()

    def fire_gather(b):
        pltpu.async_copy(tbl_sh.at[ibs[b].at[0]], bufs[b], sgs[b])

    def wait_gather(b):
        pltpu.make_async_copy(
            tbl_sh.at[ibs[b].at[0]], bufs[b], sgs[b]).wait()

    def wait_wb(b):
        pltpu.make_async_copy(
            bufs[b], out_hbm.at[pl.ds(0, CHUNK), :], sws[b]).wait()

    for b in range(3):
        fire_idx(b, b)

    @pl.when(sid < NS - 1)
    def _load_full():
        pltpu.sync_copy(tbl_hbm.at[pl.ds(sid * 640, 640), :],
                        tbl_sh.at[pl.ds(sid * 640, 640), :])

    @pl.when(sid == NS - 1)
    def _load_tail():
        pltpu.sync_copy(tbl_hbm.at[pl.ds((NS - 1) * 640, N - (NS - 1) * 640), :],
                        tbl_sh.at[pl.ds((NS - 1) * 640, N - (NS - 1) * 640), :])

    plsc.subcore_barrier()
    for b in range(2):
        wait_idx(b)
        fire_gather(b)

    def group(g, carry):
        for b in range(3):
            j = g * 3 + b
            b2 = (b + 2) % 3
            wait_gather(b)
            pltpu.async_copy(
                bufs[b], out_hbm.at[pl.ds((base + j) * CHUNK, CHUNK), :],
                sws[b])

            @pl.when(j >= 1)
            def _wait_prev_wb():
                wait_wb(b2)

            @pl.when(j + 2 < NCHT)
            def _fire_next_gather():
                wait_idx(b2)
                fire_gather(b2)

            @pl.when(j + 3 < NCHT)
            def _fire_next_idx():
                fire_idx(b, j + 3)
        return carry

    lax.fori_loop(0, NCHT // 3, group, 0)
    wait_wb((NCHT - 1) % 3)


def _gather_body(p_hbm, q_hbm, rowg_hbm, colg_hbm, tp_hbm, tq_hbm,
                 b0, b1, b2, ib0, ib1, ib2, tbl_sh,
                 sg0, sg1, sg2, sw0, sw1, sw2, si0, si1, si2):
    cid = lax.axis_index("c")
    sid = lax.axis_index("s")
    bufs = (b0, b1, b2)
    ibs = (ib0, ib1, ib2)
    sgs = (sg0, sg1, sg2)
    sws = (sw0, sw1, sw2)
    sis = (si0, si1, si2)

    @pl.when(cid == 0)
    def _core0():
        _core_gather(sid, p_hbm, tbl_sh, rowg_hbm, tp_hbm, bufs, ibs,
                     sgs, sws, sis)

    @pl.when(cid == 1)
    def _core1():
        _core_gather(sid, q_hbm, tbl_sh, colg_hbm, tq_hbm, bufs, ibs,
                     sgs, sws, sis)


@functools.cache
def _get_gather():
    return pl.kernel(
        _gather_body,
        out_type=[jax.ShapeDtypeStruct((E_HALF, D), f32),
                  jax.ShapeDtypeStruct((E_HALF, D), f32)],
        mesh=plsc.VectorSubcoreMesh(core_axis_name="c", subcore_axis_name="s"),
        scratch_types=(
            [pltpu.VMEM((CHUNK, D), f32)] * 3
            + [pltpu.VMEM((1, CHUNK), i32)] * 3
            + [pltpu.VMEM_SHARED((N, D), f32)]
            + [pltpu.SemaphoreType.DMA] * 9
        ),
    )


def _scatter_body(m0_hbm, m1_hbm, rows_hbm, out_hbm, mb0, mb1, mb2,
                  ib0, ib1, ib2, acc, sm0, sm1, sm2, si0, si1, si2,
                  sz, sa0, sa1, sa2):
    cid = lax.axis_index("c")
    sid = lax.axis_index("s")
    wid = sid * NC + cid
    mbs = (mb0, mb1, mb2)
    ibs = (ib0, ib1, ib2)
    sms = (sm0, sm1, sm2)
    sis = (si0, si1, si2)
    sas = (sa0, sa1, sa2)
    base = wid * NCH

    def fire(b, j):
        @pl.when(wid < NW // 2)
        def _fire0():
            pltpu.async_copy(
                m0_hbm.at[pl.ds((base + j) * CHUNK, CHUNK), :], mbs[b], sms[b])

        @pl.when(wid >= NW // 2)
        def _fire1():
            pltpu.async_copy(
                m1_hbm.at[pl.ds((base - NW // 2 * NCH + j) * CHUNK, CHUNK), :],
                mbs[b], sms[b])

        pltpu.async_copy(rows_hbm.at[wid, pl.ds(j, 1), :], ibs[b], sis[b])

    def zrow(r, carry):
        for c8 in range(D // 16):
            mb0[r, pl.ds(c8 * 16, 16)] = jnp.zeros((16,), f32)
        return carry

    lax.fori_loop(0, CHUNK, zrow, 0)
    tbase = sid * (ACC_ROWS // NS)
    for k in range(4):
        pltpu.async_copy(mb0, acc.at[pl.ds(tbase + k * CHUNK, CHUNK), :], sz)
    pltpu.async_copy(mb0.at[pl.ds(0, 120), :],
                     acc.at[pl.ds(tbase + 4 * CHUNK, 120), :], sz)
    fire(1, 1)
    fire(2, 2)
    for k in range(4):
        pltpu.make_async_copy(
            mb0, acc.at[pl.ds(tbase, CHUNK), :], sz).wait()
    pltpu.make_async_copy(
        mb0.at[pl.ds(0, 120), :], acc.at[pl.ds(tbase, 120), :], sz).wait()
    fire(0, 0)
    plsc.subcore_barrier()

    def group(g, carry):
        for b in range(3):
            j = g * 3 + b
            pltpu.make_async_copy(
                m0_hbm.at[pl.ds(0, CHUNK), :], mbs[b], sms[b]).wait()
            pltpu.make_async_copy(
                rows_hbm.at[wid, pl.ds(0, 1), :], ibs[b], sis[b]).wait()
            pltpu.sync_copy(mbs[b], acc.at[ibs[b].at[0]], add=True)

            @pl.when(j + 3 < NCH)
            def _refill():
                fire(b, j + 3)
        return carry

    lax.fori_loop(0, NCH // 3, group, 0)
    plsc.subcore_barrier()

    @pl.when(sid < NS - 1)
    def _copy_full():
        pltpu.sync_copy(acc.at[pl.ds(sid * 640, 640), :],
                        out_hbm.at[cid, pl.ds(sid * 640, 640), :])

    @pl.when(sid == NS - 1)
    def _copy_tail():
        pltpu.sync_copy(acc.at[pl.ds((NS - 1) * 640, N - (NS - 1) * 640), :],
                        out_hbm.at[cid, pl.ds((NS - 1) * 640, N - (NS - 1) * 640), :])


@functools.cache
def _get_scatter():
    return pl.kernel(
        _scatter_body,
        out_type=jax.ShapeDtypeStruct((NC, N, D), f32),
        mesh=plsc.VectorSubcoreMesh(core_axis_name="c", subcore_axis_name="s"),
        scratch_types=[
            pltpu.VMEM((CHUNK, D), f32), pltpu.VMEM((CHUNK, D), f32),
            pltpu.VMEM((CHUNK, D), f32),
            pltpu.VMEM((1, CHUNK), i32), pltpu.VMEM((1, CHUNK), i32),
            pltpu.VMEM((1, CHUNK), i32),
            pltpu.VMEM_SHARED((ACC_ROWS, D), f32),
            pltpu.SemaphoreType.DMA, pltpu.SemaphoreType.DMA,
            pltpu.SemaphoreType.DMA, pltpu.SemaphoreType.DMA,
            pltpu.SemaphoreType.DMA, pltpu.SemaphoreType.DMA,
            pltpu.SemaphoreType.DMA, pltpu.SemaphoreType.DMA,
            pltpu.SemaphoreType.DMA, pltpu.SemaphoreType.DMA,
        ],
    )


def _embed_tc(x_ref, ew_ref, eb_ref, w1a_ref, b1_ref, w1b_ref,
              h_ref, p_ref, q_ref):
    h = jnp.dot(x_ref[...], ew_ref[...], preferred_element_type=f32) + eb_ref[...]
    h_ref[...] = h
    p_ref[...] = jnp.dot(h, w1a_ref[...], preferred_element_type=f32) + b1_ref[...]
    q_ref[...] = jnp.dot(h, w1b_ref[...], preferred_element_type=f32)


BN = 2000
_full = lambda a, b: pl.BlockSpec((a, b), lambda j: (0, 0))
_blk = lambda: pl.BlockSpec((BN, D), lambda j: (j, 0))

_embed = pl.pallas_call(
    _embed_tc,
    grid=(N // BN,),
    in_specs=[_blk(), _full(D, D), _full(1, D), _full(D, D), _full(1, D),
              _full(D, D)],
    out_specs=[_blk(), _blk(), _blk()],
    out_shape=[jax.ShapeDtypeStruct((N, D), f32)] * 3,
)


def _edge_tc(tp_ref, tq_ref, attr_ref, w1c_ref, w2_ref, b2_ref, m_ref):
    u = tp_ref[...] + tq_ref[...] + attr_ref[...] * w1c_ref[...]
    u = u * jax.nn.sigmoid(u)
    v = jnp.dot(u.astype(bf16), w2_ref[...],
                preferred_element_type=f32) + b2_ref[...]
    m_ref[...] = v * jax.nn.sigmoid(v)


BE = 1024
_edge_mlp = pl.pallas_call(
    _edge_tc,
    grid=(E_HALF // BE,),
    in_specs=[pl.BlockSpec((BE, D), lambda j: (j, 0)),
              pl.BlockSpec((BE, D), lambda j: (j, 0)),
              pl.BlockSpec((BE, 1), lambda j: (j, 0)),
              _full(1, D), _full(D, D), _full(1, D)],
    out_specs=pl.BlockSpec((BE, D), lambda j: (j, 0)),
    out_shape=jax.ShapeDtypeStruct((E_HALF, D), f32),
)


def _node_tc(h_ref, a0_ref, a1_ref, nw1a_ref, nw1b_ref, nb1_ref, nw2_ref,
             nb2_ref, w1a_ref, b1_ref, w1b_ref, h_out, p_out, q_out):
    h = h_ref[...]
    agg = a0_ref[...] + a1_ref[...]
    t = (jnp.dot(h, nw1a_ref[...], preferred_element_type=f32)
         + jnp.dot(agg, nw1b_ref[...], preferred_element_type=f32)
         + nb1_ref[...])
    t = t * jax.nn.sigmoid(t)
    hn = jnp.dot(t, nw2_ref[...], preferred_element_type=f32) + nb2_ref[...]
    h_out[...] = hn
    p_out[...] = jnp.dot(hn, w1a_ref[...], preferred_element_type=f32) + b1_ref[...]
    q_out[...] = jnp.dot(hn, w1b_ref[...], preferred_element_type=f32)


_node = pl.pallas_call(
    _node_tc,
    grid=(N // BN,),
    in_specs=[_blk(), _blk(), _blk(),
              _full(D, D), _full(D, D), _full(1, D), _full(D, D), _full(1, D),
              _full(D, D), _full(1, D), _full(D, D)],
    out_specs=[_blk(), _blk(), _blk()],
    out_shape=[jax.ShapeDtypeStruct((N, D), f32)] * 3,
)


def _node_dec_tc(h_ref, a0_ref, a1_ref, nw1a_ref, nw1b_ref, nb1_ref,
                 nw2_ref, nb2_ref, dw1_ref, db1_ref, dw2_ref, db2_ref, o_ref):
    h = h_ref[...]
    agg = a0_ref[...] + a1_ref[...]
    t = (jnp.dot(h, nw1a_ref[...], preferred_element_type=f32)
         + jnp.dot(agg, nw1b_ref[...], preferred_element_type=f32)
         + nb1_ref[...])
    t = t * jax.nn.sigmoid(t)
    hn = jnp.dot(t, nw2_ref[...], preferred_element_type=f32) + nb2_ref[...]
    t2 = jnp.dot(hn, dw1_ref[...], preferred_element_type=f32) + db1_ref[...]
    t2 = t2 * jax.nn.sigmoid(t2)
    o_ref[...] = jnp.dot(t2, dw2_ref[...], preferred_element_type=f32) + db2_ref[...]


_node_dec = pl.pallas_call(
    _node_dec_tc,
    grid=(N // BN,),
    in_specs=[_blk(), _blk(), _blk(),
              _full(D, D), _full(D, D), _full(1, D), _full(D, D), _full(1, D),
              _full(D, D), _full(1, D), _full(D, 3), _full(1, 3)],
    out_specs=pl.BlockSpec((BN, 3), lambda j: (j, 0)),
    out_shape=jax.ShapeDtypeStruct((N, 3), f32),
)


def kernel(nodes, edges, edge_attr, emb_W, emb_b, edge_W1, edge_b1, edge_W2,
           edge_b2, node_W1, node_b1, node_W2, node_b2, dec_W1, dec_b1,
           dec_W2, dec_b2):
    row = edges[0]
    col = edges[1]
    padz = jnp.zeros((E_PAD - E,), i32)
    padn = N + jnp.arange(E_PAD - E, dtype=i32) % (ACC_ROWS - N)
    rowg = jnp.concatenate([row, padz]).reshape(2, NS, NCHT, CHUNK)
    colg = jnp.concatenate([col, padz]).reshape(2, NS, NCHT, CHUNK)
    rowsc = jnp.concatenate([row, padn]).reshape(NW, NCH, CHUNK)
    attr_p = jnp.concatenate(
        [edge_attr, jnp.zeros((E_PAD - E, 1), f32)], axis=0)
    attr_h = (attr_p[:E_HALF], attr_p[E_HALF:])

    h, p, q = _embed(nodes, emb_W, emb_b.reshape(1, D),
                     edge_W1[0, :D], edge_b1[0].reshape(1, D),
                     edge_W1[0, D:2 * D])
    out = None
    gather_fn = _get_gather()
    scatter_fn = _get_scatter()
    for i in range(NLAYERS):
        ms = []
        for half in range(2):
            tp, tq = gather_fn(p, q, rowg[half], colg[half])
            ms.append(_edge_mlp(tp, tq, attr_h[half],
                                edge_W1[i, 2 * D:].reshape(1, D),
                                edge_W2[i].astype(bf16),
                                edge_b2[i].reshape(1, D)))
        agg2 = scatter_fn(ms[0], ms[1], rowsc)
        nw1a = node_W1[i, :D]
        nw1b = node_W1[i, D:]
        if i + 1 < NLAYERS:
            h, p, q = _node(h, agg2[0], agg2[1], nw1a, nw1b,
                            node_b1[i].reshape(1, D), node_W2[i],
                            node_b2[i].reshape(1, D),
                            edge_W1[i + 1, :D], edge_b1[i + 1].reshape(1, D),
                            edge_W1[i + 1, D:2 * D])
        else:
            out = _node_dec(h, agg2[0], agg2[1], nw1a, nw1b,
                            node_b1[i].reshape(1, D), node_W2[i],
                            node_b2[i].reshape(1, D),
                            dec_W1, dec_b1.reshape(1, D),
                            dec_W2, dec_b2.reshape(1, 3))
    return out


# speedup vs baseline: 1.1669x; 1.1651x over previous
import functools

import jax
import jax.numpy as jnp
from jax import lax
from jax.experimental import pallas as pl
from jax.experimental.pallas import tpu as pltpu
from jax.experimental.pallas import tpu_sc as plsc

N = 10000
E = 320000
D = 128
NLAYERS = 4
NC, NS = 2, 16
NW = NC * NS
CHUNK = 128
NCH = 81
EW = NCH * CHUNK
E_PAD = NW * EW
E_HALF = E_PAD // 2
NCHT = E_HALF // (NS * CHUNK)
ACC_ROWS = 10112

f32 = jnp.float32
bf16 = jnp.bfloat16
i32 = jnp.int32


def _core_gather(sid, tbl_hbm, tbl_sh, idx_hbm, out_hbm, bufs, ibs,
                 sgs, sws, sis):
    base = sid * NCHT

    def fire_idx(b, j):
        pltpu.async_copy(idx_hbm.at[sid, pl.ds(j, 1), :], ibs[b], sis[b])

    def wait_idx(b):
        pltpu.make_async_copy(
            idx_hbm.at[sid, pl.ds(0, 1), :], ibs[b], sis[b]).wait()

    def fire_gather(b):
        pltpu.async_copy(tbl_sh.at[ibs[b].at[0]], bufs[b], sgs[b])

    def wait_gather(b):
        pltpu.make_async_copy(
            tbl_sh.at[ibs[b].at[0]], bufs[b], sgs[b]).wait()

    def wait_wb(b):
        pltpu.make_async_copy(
            bufs[b], out_hbm.at[pl.ds(0, CHUNK), :], sws[b]).wait()

    for b in range(3):
        fire_idx(b, b)

    @pl.when(sid < NS - 1)
    def _load_full():
        pltpu.sync_copy(tbl_hbm.at[pl.ds(sid * 640, 640), :],
                        tbl_sh.at[pl.ds(sid * 640, 640), :])

    @pl.when(sid == NS - 1)
    def _load_tail():
        pltpu.sync_copy(tbl_hbm.at[pl.ds((NS - 1) * 640, N - (NS - 1) * 640), :],
                        tbl_sh.at[pl.ds((NS - 1) * 640, N - (NS - 1) * 640), :])

    plsc.subcore_barrier()
    for b in range(2):
        wait_idx(b)
        fire_gather(b)

    def group(g, carry):
        for b in range(3):
            j = g * 3 + b
            b2 = (b + 2) % 3
            wait_gather(b)
            pltpu.async_copy(
                bufs[b], out_hbm.at[pl.ds((base + j) * CHUNK, CHUNK), :],
                sws[b])

            @pl.when(j >= 1)
            def _wait_prev_wb():
                wait_wb(b2)

            @pl.when(j + 2 < NCHT)
            def _fire_next_gather():
                wait_idx(b2)
                fire_gather(b2)

            @pl.when(j + 3 < NCHT)
            def _fire_next_idx():
                fire_idx(b, j + 3)
        return carry

    lax.fori_loop(0, NCHT // 3, group, 0)
    wait_wb((NCHT - 1) % 3)


def _gather_body(p_hbm, q_hbm, rowg_hbm, colg_hbm, tp_hbm, tq_hbm,
                 b0, b1, b2, ib0, ib1, ib2, tbl_sh,
                 sg0, sg1, sg2, sw0, sw1, sw2, si0, si1, si2):
    cid = lax.axis_index("c")
    sid = lax.axis_index("s")
    bufs = (b0, b1, b2)
    ibs = (ib0, ib1, ib2)
    sgs = (sg0, sg1, sg2)
    sws = (sw0, sw1, sw2)
    sis = (si0, si1, si2)

    @pl.when(cid == 0)
    def _core0():
        _core_gather(sid, p_hbm, tbl_sh, rowg_hbm, tp_hbm, bufs, ibs,
                     sgs, sws, sis)

    @pl.when(cid == 1)
    def _core1():
        _core_gather(sid, q_hbm, tbl_sh, colg_hbm, tq_hbm, bufs, ibs,
                     sgs, sws, sis)


@functools.cache
def _get_gather():
    return pl.kernel(
        _gather_body,
        out_type=[jax.ShapeDtypeStruct((E_HALF, D), f32),
                  jax.ShapeDtypeStruct((E_HALF, D), f32)],
        mesh=plsc.VectorSubcoreMesh(core_axis_name="c", subcore_axis_name="s"),
        scratch_types=(
            [pltpu.VMEM((CHUNK, D), f32)] * 3
            + [pltpu.VMEM((1, CHUNK), i32)] * 3
            + [pltpu.VMEM_SHARED((N, D), f32)]
            + [pltpu.SemaphoreType.DMA] * 9
        ),
    )


def _scatter_body(m0_hbm, m1_hbm, rows_hbm, out_hbm, mb0, mb1, mb2,
                  ib0, ib1, ib2, acc, sm0, sm1, sm2, si0, si1, si2,
                  sz, sa0, sa1, sa2):
    cid = lax.axis_index("c")
    sid = lax.axis_index("s")
    wid = sid * NC + cid
    mbs = (mb0, mb1, mb2)
    ibs = (ib0, ib1, ib2)
    sms = (sm0, sm1, sm2)
    sis = (si0, si1, si2)
    sas = (sa0, sa1, sa2)
    base = wid * NCH

    def fire(b, j):
        @pl.when(wid < NW // 2)
        def _fire0():
            pltpu.async_copy(
                m0_hbm.at[pl.ds((base + j) * CHUNK, CHUNK), :], mbs[b], sms[b])

        @pl.when(wid >= NW // 2)
        def _fire1():
            pltpu.async_copy(
                m1_hbm.at[pl.ds((base - NW // 2 * NCH + j) * CHUNK, CHUNK), :],
                mbs[b], sms[b])

        pltpu.async_copy(rows_hbm.at[wid, pl.ds(j, 1), :], ibs[b], sis[b])

    def zrow(r, carry):
        for c8 in range(D // 16):
            mb0[r, pl.ds(c8 * 16, 16)] = jnp.zeros((16,), f32)
        return carry

    lax.fori_loop(0, CHUNK, zrow, 0)
    tbase = sid * (ACC_ROWS // NS)
    for k in range(4):
        pltpu.async_copy(mb0, acc.at[pl.ds(tbase + k * CHUNK, CHUNK), :], sz)
    pltpu.async_copy(mb0.at[pl.ds(0, 120), :],
                     acc.at[pl.ds(tbase + 4 * CHUNK, 120), :], sz)
    fire(1, 1)
    fire(2, 2)
    for k in range(4):
        pltpu.make_async_copy(
            mb0, acc.at[pl.ds(tbase, CHUNK), :], sz).wait()
    pltpu.make_async_copy(
        mb0.at[pl.ds(0, 120), :], acc.at[pl.ds(tbase, 120), :], sz).wait()
    fire(0, 0)
    plsc.subcore_barrier()

    def group(g, carry):
        for b in range(3):
            j = g * 3 + b
            pltpu.make_async_copy(
                m0_hbm.at[pl.ds(0, CHUNK), :], mbs[b], sms[b]).wait()
            pltpu.make_async_copy(
                rows_hbm.at[wid, pl.ds(0, 1), :], ibs[b], sis[b]).wait()
            pltpu.sync_copy(mbs[b], acc.at[ibs[b].at[0]], add=True)

            @pl.when(j + 3 < NCH)
            def _refill():
                fire(b, j + 3)
        return carry

    lax.fori_loop(0, NCH // 3, group, 0)
    plsc.subcore_barrier()

    @pl.when(sid < NS - 1)
    def _copy_full():
        pltpu.sync_copy(acc.at[pl.ds(sid * 640, 640), :],
                        out_hbm.at[cid, pl.ds(sid * 640, 640), :])

    @pl.when(sid == NS - 1)
    def _copy_tail():
        pltpu.sync_copy(acc.at[pl.ds((NS - 1) * 640, N - (NS - 1) * 640), :],
                        out_hbm.at[cid, pl.ds((NS - 1) * 640, N - (NS - 1) * 640), :])


@functools.cache
def _get_scatter():
    return pl.kernel(
        _scatter_body,
        out_type=jax.ShapeDtypeStruct((NC, N, D), f32),
        mesh=plsc.VectorSubcoreMesh(core_axis_name="c", subcore_axis_name="s"),
        scratch_types=[
            pltpu.VMEM((CHUNK, D), f32), pltpu.VMEM((CHUNK, D), f32),
            pltpu.VMEM((CHUNK, D), f32),
            pltpu.VMEM((1, CHUNK), i32), pltpu.VMEM((1, CHUNK), i32),
            pltpu.VMEM((1, CHUNK), i32),
            pltpu.VMEM_SHARED((ACC_ROWS, D), f32),
            pltpu.SemaphoreType.DMA, pltpu.SemaphoreType.DMA,
            pltpu.SemaphoreType.DMA, pltpu.SemaphoreType.DMA,
            pltpu.SemaphoreType.DMA, pltpu.SemaphoreType.DMA,
            pltpu.SemaphoreType.DMA, pltpu.SemaphoreType.DMA,
            pltpu.SemaphoreType.DMA, pltpu.SemaphoreType.DMA,
        ],
    )


def _embed_tc(x_ref, ew_ref, eb_ref, w1a_ref, b1_ref, w1b_ref,
              h_ref, p_ref, q_ref):
    h = jnp.dot(x_ref[...], ew_ref[...], preferred_element_type=f32) + eb_ref[...]
    h_ref[...] = h
    p_ref[...] = jnp.dot(h, w1a_ref[...], preferred_element_type=f32) + b1_ref[...]
    q_ref[...] = jnp.dot(h, w1b_ref[...], preferred_element_type=f32)


BN = 2000
_full = lambda a, b: pl.BlockSpec((a, b), lambda j: (0, 0))
_blk = lambda: pl.BlockSpec((BN, D), lambda j: (j, 0))

_embed = pl.pallas_call(
    _embed_tc,
    grid=(N // BN,),
    in_specs=[_blk(), _full(D, D), _full(1, D), _full(D, D), _full(1, D),
              _full(D, D)],
    out_specs=[_blk(), _blk(), _blk()],
    out_shape=[jax.ShapeDtypeStruct((N, D), f32)] * 3,
)


def _edge_tc(tp_ref, tq_ref, attr_ref, w1c_ref, w2_ref, b2_ref, m_ref):
    u = tp_ref[...] + tq_ref[...] + attr_ref[...] * w1c_ref[...]
    u = u * jax.nn.sigmoid(u)
    v = jnp.dot(u.astype(bf16), w2_ref[...],
                preferred_element_type=f32) + b2_ref[...]
    m_ref[...] = v * jax.nn.sigmoid(v)


BE = 2048
_edge_mlp = pl.pallas_call(
    _edge_tc,
    grid=(E_HALF // BE,),
    in_specs=[pl.BlockSpec((BE, D), lambda j: (j, 0)),
              pl.BlockSpec((BE, D), lambda j: (j, 0)),
              pl.BlockSpec((BE, 1), lambda j: (j, 0)),
              _full(1, D), _full(D, D), _full(1, D)],
    out_specs=pl.BlockSpec((BE, D), lambda j: (j, 0)),
    out_shape=jax.ShapeDtypeStruct((E_HALF, D), f32),
)


def _node_tc(h_ref, a0_ref, a1_ref, nw1a_ref, nw1b_ref, nb1_ref, nw2_ref,
             nb2_ref, w1a_ref, b1_ref, w1b_ref, h_out, p_out, q_out):
    h = h_ref[...]
    agg = a0_ref[...] + a1_ref[...]
    t = (jnp.dot(h, nw1a_ref[...], preferred_element_type=f32)
         + jnp.dot(agg, nw1b_ref[...], preferred_element_type=f32)
         + nb1_ref[...])
    t = t * jax.nn.sigmoid(t)
    hn = jnp.dot(t, nw2_ref[...], preferred_element_type=f32) + nb2_ref[...]
    h_out[...] = hn
    p_out[...] = jnp.dot(hn, w1a_ref[...], preferred_element_type=f32) + b1_ref[...]
    q_out[...] = jnp.dot(hn, w1b_ref[...], preferred_element_type=f32)


_node = pl.pallas_call(
    _node_tc,
    grid=(N // BN,),
    in_specs=[_blk(), _blk(), _blk(),
              _full(D, D), _full(D, D), _full(1, D), _full(D, D), _full(1, D),
              _full(D, D), _full(1, D), _full(D, D)],
    out_specs=[_blk(), _blk(), _blk()],
    out_shape=[jax.ShapeDtypeStruct((N, D), f32)] * 3,
)


def _node_dec_tc(h_ref, a0_ref, a1_ref, nw1a_ref, nw1b_ref, nb1_ref,
                 nw2_ref, nb2_ref, dw1_ref, db1_ref, dw2_ref, db2_ref, o_ref):
    h = h_ref[...]
    agg = a0_ref[...] + a1_ref[...]
    t = (jnp.dot(h, nw1a_ref[...], preferred_element_type=f32)
         + jnp.dot(agg, nw1b_ref[...], preferred_element_type=f32)
         + nb1_ref[...])
    t = t * jax.nn.sigmoid(t)
    hn = jnp.dot(t, nw2_ref[...], preferred_element_type=f32) + nb2_ref[...]
    t2 = jnp.dot(hn, dw1_ref[...], preferred_element_type=f32) + db1_ref[...]
    t2 = t2 * jax.nn.sigmoid(t2)
    o_ref[...] = jnp.dot(t2, dw2_ref[...], preferred_element_type=f32) + db2_ref[...]


_node_dec = pl.pallas_call(
    _node_dec_tc,
    grid=(N // BN,),
    in_specs=[_blk(), _blk(), _blk(),
              _full(D, D), _full(D, D), _full(1, D), _full(D, D), _full(1, D),
              _full(D, D), _full(1, D), _full(D, 3), _full(1, 3)],
    out_specs=pl.BlockSpec((BN, 3), lambda j: (j, 0)),
    out_shape=jax.ShapeDtypeStruct((N, 3), f32),
)


def kernel(nodes, edges, edge_attr, emb_W, emb_b, edge_W1, edge_b1, edge_W2,
           edge_b2, node_W1, node_b1, node_W2, node_b2, dec_W1, dec_b1,
           dec_W2, dec_b2):
    row = edges[0]
    col = edges[1]
    padz = jnp.zeros((E_PAD - E,), i32)
    padn = N + jnp.arange(E_PAD - E, dtype=i32) % (ACC_ROWS - N)
    rowg = jnp.concatenate([row, padz]).reshape(2, NS, NCHT, CHUNK)
    colg = jnp.concatenate([col, padz]).reshape(2, NS, NCHT, CHUNK)
    rowsc = jnp.concatenate([row, padn]).reshape(NW, NCH, CHUNK)
    attr_p = jnp.concatenate(
        [edge_attr, jnp.zeros((E_PAD - E, 1), f32)], axis=0)
    attr_h = (attr_p[:E_HALF], attr_p[E_HALF:])

    h, p, q = _embed(nodes, emb_W, emb_b.reshape(1, D),
                     edge_W1[0, :D], edge_b1[0].reshape(1, D),
                     edge_W1[0, D:2 * D])
    out = None
    gather_fn = _get_gather()
    scatter_fn = _get_scatter()
    for i in range(NLAYERS):
        ms = []
        for half in range(2):
            tp, tq = gather_fn(p, q, rowg[half], colg[half])
            ms.append(_edge_mlp(tp, tq, attr_h[half],
                                edge_W1[i, 2 * D:].reshape(1, D),
                                edge_W2[i].astype(bf16),
                                edge_b2[i].reshape(1, D)))
        agg2 = scatter_fn(ms[0], ms[1], rowsc)
        nw1a = node_W1[i, :D]
        nw1b = node_W1[i, D:]
        if i + 1 < NLAYERS:
            h, p, q = _node(h, agg2[0], agg2[1], nw1a, nw1b,
                            node_b1[i].reshape(1, D), node_W2[i],
                            node_b2[i].reshape(1, D),
                            edge_W1[i + 1, :D], edge_b1[i + 1].reshape(1, D),
                            edge_W1[i + 1, D:2 * D])
        else:
            out = _node_dec(h, agg2[0], agg2[1], nw1a, nw1b,
                            node_b1[i].reshape(1, D), node_W2[i],
                            node_b2[i].reshape(1, D),
                            dec_W1, dec_b1.reshape(1, D),
                            dec_W2, dec_b2.reshape(1, 3))
    return out


# speedup vs baseline: 1.2232x; 1.0483x over previous
import functools

import jax
import jax.numpy as jnp
from jax import lax
from jax.experimental import pallas as pl
from jax.experimental.pallas import tpu as pltpu
from jax.experimental.pallas import tpu_sc as plsc

N = 10000
E = 320000
D = 128
NLAYERS = 4
NC, NS = 2, 16
NW = NC * NS
CHUNK = 128
NCH = 81
EW = NCH * CHUNK
E_PAD = NW * EW
E_HALF = E_PAD // 2
NCHT = E_HALF // (NS * CHUNK)
ACC_ROWS = 10112

f32 = jnp.float32
bf16 = jnp.bfloat16
i32 = jnp.int32


def _core_gather(sid, tbl_hbm, tbl_sh, idx_hbm, out_hbm, bufs, ibs,
                 sgs, sws, sis):
    base = sid * NCHT

    def fire_idx(b, j):
        pltpu.async_copy(idx_hbm.at[sid, pl.ds(j, 1), :], ibs[b], sis[b])

    def wait_idx(b):
        pltpu.make_async_copy(
            idx_hbm.at[sid, pl.ds(0, 1), :], ibs[b], sis[b]).wait()

    def fire_gather(b):
        pltpu.async_copy(tbl_sh.at[ibs[b].at[0]], bufs[b], sgs[b])

    def wait_gather(b):
        pltpu.make_async_copy(
            tbl_sh.at[ibs[b].at[0]], bufs[b], sgs[b]).wait()

    def wait_wb(b):
        pltpu.make_async_copy(
            bufs[b], out_hbm.at[pl.ds(0, CHUNK), :], sws[b]).wait()

    for b in range(3):
        fire_idx(b, b)

    @pl.when(sid < NS - 1)
    def _load_full():
        pltpu.sync_copy(tbl_hbm.at[pl.ds(sid * 640, 640), :],
                        tbl_sh.at[pl.ds(sid * 640, 640), :])

    @pl.when(sid == NS - 1)
    def _load_tail():
        pltpu.sync_copy(tbl_hbm.at[pl.ds((NS - 1) * 640, N - (NS - 1) * 640), :],
                        tbl_sh.at[pl.ds((NS - 1) * 640, N - (NS - 1) * 640), :])

    plsc.subcore_barrier()
    for b in range(2):
        wait_idx(b)
        fire_gather(b)

    def group(g, carry):
        for b in range(3):
            j = g * 3 + b
            b2 = (b + 2) % 3
            wait_gather(b)
            pltpu.async_copy(
                bufs[b], out_hbm.at[pl.ds((base + j) * CHUNK, CHUNK), :],
                sws[b])

            @pl.when(j >= 1)
            def _wait_prev_wb():
                wait_wb(b2)

            @pl.when(j + 2 < NCHT)
            def _fire_next_gather():
                wait_idx(b2)
                fire_gather(b2)

            @pl.when(j + 3 < NCHT)
            def _fire_next_idx():
                fire_idx(b, j + 3)
        return carry

    lax.fori_loop(0, NCHT // 3, group, 0)
    wait_wb((NCHT - 1) % 3)


def _gather_body(p_hbm, q_hbm, rowg_hbm, colg_hbm, tp_hbm, tq_hbm,
                 b0, b1, b2, ib0, ib1, ib2, tbl_sh,
                 sg0, sg1, sg2, sw0, sw1, sw2, si0, si1, si2):
    cid = lax.axis_index("c")
    sid = lax.axis_index("s")
    bufs = (b0, b1, b2)
    ibs = (ib0, ib1, ib2)
    sgs = (sg0, sg1, sg2)
    sws = (sw0, sw1, sw2)
    sis = (si0, si1, si2)

    @pl.when(cid == 0)
    def _core0():
        _core_gather(sid, p_hbm, tbl_sh, rowg_hbm, tp_hbm, bufs, ibs,
                     sgs, sws, sis)

    @pl.when(cid == 1)
    def _core1():
        _core_gather(sid, q_hbm, tbl_sh, colg_hbm, tq_hbm, bufs, ibs,
                     sgs, sws, sis)


@functools.cache
def _get_gather():
    return pl.kernel(
        _gather_body,
        out_type=[jax.ShapeDtypeStruct((E_HALF, D), f32),
                  jax.ShapeDtypeStruct((E_HALF, D), f32)],
        mesh=plsc.VectorSubcoreMesh(core_axis_name="c", subcore_axis_name="s"),
        scratch_types=(
            [pltpu.VMEM((CHUNK, D), f32)] * 3
            + [pltpu.VMEM((1, CHUNK), i32)] * 3
            + [pltpu.VMEM_SHARED((N, D), f32)]
            + [pltpu.SemaphoreType.DMA] * 9
        ),
    )


def _scatter_body(m0_hbm, m1_hbm, rows_hbm, out_hbm, mb0, mb1, mb2,
                  ib0, ib1, ib2, acc, sm0, sm1, sm2, si0, si1, si2,
                  sz, sa0, sa1, sa2):
    cid = lax.axis_index("c")
    sid = lax.axis_index("s")
    wid = sid * NC + cid
    mbs = (mb0, mb1, mb2)
    ibs = (ib0, ib1, ib2)
    sms = (sm0, sm1, sm2)
    sis = (si0, si1, si2)
    sas = (sa0, sa1, sa2)
    base = wid * NCH

    def fire(b, j):
        @pl.when(wid < NW // 2)
        def _fire0():
            pltpu.async_copy(
                m0_hbm.at[pl.ds((base + j) * CHUNK, CHUNK), :], mbs[b], sms[b])

        @pl.when(wid >= NW // 2)
        def _fire1():
            pltpu.async_copy(
                m1_hbm.at[pl.ds((base - NW // 2 * NCH + j) * CHUNK, CHUNK), :],
                mbs[b], sms[b])

        pltpu.async_copy(rows_hbm.at[wid, pl.ds(j, 1), :], ibs[b], sis[b])

    def zrow(r, carry):
        for c8 in range(D // 16):
            mb0[r, pl.ds(c8 * 16, 16)] = jnp.zeros((16,), f32)
        return carry

    lax.fori_loop(0, CHUNK, zrow, 0)
    tbase = sid * (ACC_ROWS // NS)
    for k in range(4):
        pltpu.async_copy(mb0, acc.at[pl.ds(tbase + k * CHUNK, CHUNK), :], sz)
    pltpu.async_copy(mb0.at[pl.ds(0, 120), :],
                     acc.at[pl.ds(tbase + 4 * CHUNK, 120), :], sz)
    fire(1, 1)
    fire(2, 2)
    for k in range(4):
        pltpu.make_async_copy(
            mb0, acc.at[pl.ds(tbase, CHUNK), :], sz).wait()
    pltpu.make_async_copy(
        mb0.at[pl.ds(0, 120), :], acc.at[pl.ds(tbase, 120), :], sz).wait()
    fire(0, 0)
    plsc.subcore_barrier()

    def group(g, carry):
        for b in range(3):
            j = g * 3 + b
            pltpu.make_async_copy(
                m0_hbm.at[pl.ds(0, CHUNK), :], mbs[b], sms[b]).wait()
            pltpu.make_async_copy(
                rows_hbm.at[wid, pl.ds(0, 1), :], ibs[b], sis[b]).wait()
            pltpu.sync_copy(mbs[b], acc.at[ibs[b].at[0]], add=True)

            @pl.when(j + 3 < NCH)
            def _refill():
                fire(b, j + 3)
        return carry

    lax.fori_loop(0, NCH // 3, group, 0)
    plsc.subcore_barrier()

    @pl.when(sid < NS - 1)
    def _copy_full():
        pltpu.sync_copy(acc.at[pl.ds(sid * 640, 640), :],
                        out_hbm.at[cid, pl.ds(sid * 640, 640), :])

    @pl.when(sid == NS - 1)
    def _copy_tail():
        pltpu.sync_copy(acc.at[pl.ds((NS - 1) * 640, N - (NS - 1) * 640), :],
                        out_hbm.at[cid, pl.ds((NS - 1) * 640, N - (NS - 1) * 640), :])


@functools.cache
def _get_scatter():
    return pl.kernel(
        _scatter_body,
        out_type=jax.ShapeDtypeStruct((NC, N, D), f32),
        mesh=plsc.VectorSubcoreMesh(core_axis_name="c", subcore_axis_name="s"),
        scratch_types=[
            pltpu.VMEM((CHUNK, D), f32), pltpu.VMEM((CHUNK, D), f32),
            pltpu.VMEM((CHUNK, D), f32),
            pltpu.VMEM((1, CHUNK), i32), pltpu.VMEM((1, CHUNK), i32),
            pltpu.VMEM((1, CHUNK), i32),
            pltpu.VMEM_SHARED((ACC_ROWS, D), f32),
            pltpu.SemaphoreType.DMA, pltpu.SemaphoreType.DMA,
            pltpu.SemaphoreType.DMA, pltpu.SemaphoreType.DMA,
            pltpu.SemaphoreType.DMA, pltpu.SemaphoreType.DMA,
            pltpu.SemaphoreType.DMA, pltpu.SemaphoreType.DMA,
            pltpu.SemaphoreType.DMA, pltpu.SemaphoreType.DMA,
        ],
    )


def _embed_tc(x_ref, ew_ref, eb_ref, w1a_ref, b1_ref, w1b_ref,
              h_ref, p_ref, q_ref):
    h = jnp.dot(x_ref[...], ew_ref[...], preferred_element_type=f32) + eb_ref[...]
    h_ref[...] = h
    p_ref[...] = jnp.dot(h, w1a_ref[...], preferred_element_type=f32) + b1_ref[...]
    q_ref[...] = jnp.dot(h, w1b_ref[...], preferred_element_type=f32)


BN = 2000
_full = lambda a, b: pl.BlockSpec((a, b), lambda j: (0, 0))
_blk = lambda: pl.BlockSpec((BN, D), lambda j: (j, 0))

_embed = pl.pallas_call(
    _embed_tc,
    grid=(N // BN,),
    in_specs=[_blk(), _full(D, D), _full(1, D), _full(D, D), _full(1, D),
              _full(D, D)],
    out_specs=[_blk(), _blk(), _blk()],
    out_shape=[jax.ShapeDtypeStruct((N, D), f32)] * 3,
)


def _edge_tc(tp_ref, tq_ref, attr_ref, w1c_ref, w2_ref, b2_ref, m_ref):
    u = tp_ref[...] + tq_ref[...] + attr_ref[...] * w1c_ref[...]
    u = u * jax.nn.sigmoid(u)
    v = jnp.dot(u.astype(bf16), w2_ref[...],
                preferred_element_type=f32) + b2_ref[...]
    m_ref[...] = v * jax.nn.sigmoid(v)


BE = 3072
_edge_mlp = pl.pallas_call(
    _edge_tc,
    grid=(E_HALF // BE,),
    in_specs=[pl.BlockSpec((BE, D), lambda j: (j, 0)),
              pl.BlockSpec((BE, D), lambda j: (j, 0)),
              pl.BlockSpec((BE, 1), lambda j: (j, 0)),
              _full(1, D), _full(D, D), _full(1, D)],
    out_specs=pl.BlockSpec((BE, D), lambda j: (j, 0)),
    out_shape=jax.ShapeDtypeStruct((E_HALF, D), f32),
)


def _node_tc(h_ref, a0_ref, a1_ref, nw1a_ref, nw1b_ref, nb1_ref, nw2_ref,
             nb2_ref, w1a_ref, b1_ref, w1b_ref, h_out, p_out, q_out):
    h = h_ref[...]
    agg = a0_ref[...] + a1_ref[...]
    t = (jnp.dot(h, nw1a_ref[...], preferred_element_type=f32)
         + jnp.dot(agg, nw1b_ref[...], preferred_element_type=f32)
         + nb1_ref[...])
    t = t * jax.nn.sigmoid(t)
    hn = jnp.dot(t, nw2_ref[...], preferred_element_type=f32) + nb2_ref[...]
    h_out[...] = hn
    p_out[...] = jnp.dot(hn, w1a_ref[...], preferred_element_type=f32) + b1_ref[...]
    q_out[...] = jnp.dot(hn, w1b_ref[...], preferred_element_type=f32)


_node = pl.pallas_call(
    _node_tc,
    grid=(N // BN,),
    in_specs=[_blk(), _blk(), _blk(),
              _full(D, D), _full(D, D), _full(1, D), _full(D, D), _full(1, D),
              _full(D, D), _full(1, D), _full(D, D)],
    out_specs=[_blk(), _blk(), _blk()],
    out_shape=[jax.ShapeDtypeStruct((N, D), f32)] * 3,
)


def _node_dec_tc(h_ref, a0_ref, a1_ref, nw1a_ref, nw1b_ref, nb1_ref,
                 nw2_ref, nb2_ref, dw1_ref, db1_ref, dw2_ref, db2_ref, o_ref):
    h = h_ref[...]
    agg = a0_ref[...] + a1_ref[...]
    t = (jnp.dot(h, nw1a_ref[...], preferred_element_type=f32)
         + jnp.dot(agg, nw1b_ref[...], preferred_element_type=f32)
         + nb1_ref[...])
    t = t * jax.nn.sigmoid(t)
    hn = jnp.dot(t, nw2_ref[...], preferred_element_type=f32) + nb2_ref[...]
    t2 = jnp.dot(hn, dw1_ref[...], preferred_element_type=f32) + db1_ref[...]
    t2 = t2 * jax.nn.sigmoid(t2)
    o_ref[...] = jnp.dot(t2, dw2_ref[...], preferred_element_type=f32) + db2_ref[...]


_node_dec = pl.pallas_call(
    _node_dec_tc,
    grid=(N // BN,),
    in_specs=[_blk(), _blk(), _blk(),
              _full(D, D), _full(D, D), _full(1, D), _full(D, D), _full(1, D),
              _full(D, D), _full(1, D), _full(D, 3), _full(1, 3)],
    out_specs=pl.BlockSpec((BN, 3), lambda j: (j, 0)),
    out_shape=jax.ShapeDtypeStruct((N, 3), f32),
)


def kernel(nodes, edges, edge_attr, emb_W, emb_b, edge_W1, edge_b1, edge_W2,
           edge_b2, node_W1, node_b1, node_W2, node_b2, dec_W1, dec_b1,
           dec_W2, dec_b2):
    row = edges[0]
    col = edges[1]
    padz = jnp.zeros((E_PAD - E,), i32)
    padn = N + jnp.arange(E_PAD - E, dtype=i32) % (ACC_ROWS - N)
    rowg = jnp.concatenate([row, padz]).reshape(2, NS, NCHT, CHUNK)
    colg = jnp.concatenate([col, padz]).reshape(2, NS, NCHT, CHUNK)
    rowsc = jnp.concatenate([row, padn]).reshape(NW, NCH, CHUNK)
    attr_p = jnp.concatenate(
        [edge_attr, jnp.zeros((E_PAD - E, 1), f32)], axis=0)
    attr_h = (attr_p[:E_HALF], attr_p[E_HALF:])

    h, p, q = _embed(nodes, emb_W, emb_b.reshape(1, D),
                     edge_W1[0, :D], edge_b1[0].reshape(1, D),
                     edge_W1[0, D:2 * D])
    out = None
    gather_fn = _get_gather()
    scatter_fn = _get_scatter()
    for i in range(NLAYERS):
        ms = []
        for half in range(2):
            tp, tq = gather_fn(p, q, rowg[half], colg[half])
            ms.append(_edge_mlp(tp, tq, attr_h[half],
                                edge_W1[i, 2 * D:].reshape(1, D),
                                edge_W2[i].astype(bf16),
                                edge_b2[i].reshape(1, D)))
        agg2 = scatter_fn(ms[0], ms[1], rowsc)
        nw1a = node_W1[i, :D]
        nw1b = node_W1[i, D:]
        if i + 1 < NLAYERS:
            h, p, q = _node(h, agg2[0], agg2[1], nw1a, nw1b,
                            node_b1[i].reshape(1, D), node_W2[i],
                            node_b2[i].reshape(1, D),
                            edge_W1[i + 1, :D], edge_b1[i + 1].reshape(1, D),
                            edge_W1[i + 1, D:2 * D])
        else:
            out = _node_dec(h, agg2[0], agg2[1], nw1a, nw1b,
                            node_b1[i].reshape(1, D), node_W2[i],
                            node_b2[i].reshape(1, D),
                            dec_W1, dec_b1.reshape(1, D),
                            dec_W2, dec_b2.reshape(1, 3))
    return out


# speedup vs baseline: 1.2418x; 1.0152x over previous
import functools

import jax
import jax.numpy as jnp
from jax import lax
from jax.experimental import pallas as pl
from jax.experimental.pallas import tpu as pltpu
from jax.experimental.pallas import tpu_sc as plsc

N = 10000
E = 320000
D = 128
NLAYERS = 4
NC, NS = 2, 16
NW = NC * NS
CHUNK = 128
NCH = 81
EW = NCH * CHUNK
E_PAD = NW * EW
E_HALF = E_PAD // 2
NCHT = E_HALF // (NS * CHUNK)
ACC_ROWS = 10112

f32 = jnp.float32
bf16 = jnp.bfloat16
i32 = jnp.int32


def _core_gather(sid, tbl_hbm, tbl_sh, idx_hbm, out_hbm, bufs, ibs,
                 sgs, sws, sis):
    base = sid * NCHT

    def fire_idx(b, j):
        pltpu.async_copy(idx_hbm.at[sid, pl.ds(j, 1), :], ibs[b], sis[b])

    def wait_idx(b):
        pltpu.make_async_copy(
            idx_hbm.at[sid, pl.ds(0, 1), :], ibs[b], sis[b]).wait()

    def fire_gather(b):
        pltpu.async_copy(tbl_sh.at[ibs[b].at[0]], bufs[b], sgs[b])

    def wait_gather(b):
        pltpu.make_async_copy(
            tbl_sh.at[ibs[b].at[0]], bufs[b], sgs[b]).wait()

    def wait_wb(b):
        pltpu.make_async_copy(
            bufs[b], out_hbm.at[pl.ds(0, CHUNK), :], sws[b]).wait()

    for b in range(3):
        fire_idx(b, b)

    @pl.when(sid < NS - 1)
    def _load_full():
        pltpu.sync_copy(tbl_hbm.at[pl.ds(sid * 640, 640), :],
                        tbl_sh.at[pl.ds(sid * 640, 640), :])

    @pl.when(sid == NS - 1)
    def _load_tail():
        pltpu.sync_copy(tbl_hbm.at[pl.ds((NS - 1) * 640, N - (NS - 1) * 640), :],
                        tbl_sh.at[pl.ds((NS - 1) * 640, N - (NS - 1) * 640), :])

    plsc.subcore_barrier()
    for b in range(2):
        wait_idx(b)
        fire_gather(b)

    def group(g, carry):
        for b in range(3):
            j = g * 3 + b
            b2 = (b + 2) % 3
            wait_gather(b)
            pltpu.async_copy(
                bufs[b], out_hbm.at[pl.ds((base + j) * CHUNK, CHUNK), :],
                sws[b])

            @pl.when(j >= 1)
            def _wait_prev_wb():
                wait_wb(b2)

            @pl.when(j + 2 < NCHT)
            def _fire_next_gather():
                wait_idx(b2)
                fire_gather(b2)

            @pl.when(j + 3 < NCHT)
            def _fire_next_idx():
                fire_idx(b, j + 3)
        return carry

    lax.fori_loop(0, NCHT // 3, group, 0)
    wait_wb((NCHT - 1) % 3)


def _gather_body(p_hbm, q_hbm, rowg_hbm, colg_hbm, tp_hbm, tq_hbm,
                 b0, b1, b2, ib0, ib1, ib2, tbl_sh,
                 sg0, sg1, sg2, sw0, sw1, sw2, si0, si1, si2):
    cid = lax.axis_index("c")
    sid = lax.axis_index("s")
    bufs = (b0, b1, b2)
    ibs = (ib0, ib1, ib2)
    sgs = (sg0, sg1, sg2)
    sws = (sw0, sw1, sw2)
    sis = (si0, si1, si2)

    @pl.when(cid == 0)
    def _core0():
        _core_gather(sid, p_hbm, tbl_sh, rowg_hbm, tp_hbm, bufs, ibs,
                     sgs, sws, sis)

    @pl.when(cid == 1)
    def _core1():
        _core_gather(sid, q_hbm, tbl_sh, colg_hbm, tq_hbm, bufs, ibs,
                     sgs, sws, sis)


@functools.cache
def _get_gather():
    return pl.kernel(
        _gather_body,
        out_type=[jax.ShapeDtypeStruct((E_HALF, D), f32),
                  jax.ShapeDtypeStruct((E_HALF, D), f32)],
        mesh=plsc.VectorSubcoreMesh(core_axis_name="c", subcore_axis_name="s"),
        scratch_types=(
            [pltpu.VMEM((CHUNK, D), f32)] * 3
            + [pltpu.VMEM((1, CHUNK), i32)] * 3
            + [pltpu.VMEM_SHARED((N, D), f32)]
            + [pltpu.SemaphoreType.DMA] * 9
        ),
    )


def _scatter_body(m0_hbm, m1_hbm, rows_hbm, out_hbm, mb0, mb1, mb2,
                  ib0, ib1, ib2, acc, sm0, sm1, sm2, si0, si1, si2,
                  sz, sa0, sa1, sa2):
    cid = lax.axis_index("c")
    sid = lax.axis_index("s")
    wid = sid * NC + cid
    mbs = (mb0, mb1, mb2)
    ibs = (ib0, ib1, ib2)
    sms = (sm0, sm1, sm2)
    sis = (si0, si1, si2)
    sas = (sa0, sa1, sa2)
    base = wid * NCH

    def fire(b, j):
        @pl.when(wid < NW // 2)
        def _fire0():
            pltpu.async_copy(
                m0_hbm.at[pl.ds((base + j) * CHUNK, CHUNK), :], mbs[b], sms[b])

        @pl.when(wid >= NW // 2)
        def _fire1():
            pltpu.async_copy(
                m1_hbm.at[pl.ds((base - NW // 2 * NCH + j) * CHUNK, CHUNK), :],
                mbs[b], sms[b])

        pltpu.async_copy(rows_hbm.at[wid, pl.ds(j, 1), :], ibs[b], sis[b])

    def zrow(r, carry):
        for c8 in range(D // 16):
            mb0[r, pl.ds(c8 * 16, 16)] = jnp.zeros((16,), f32)
        return carry

    lax.fori_loop(0, CHUNK, zrow, 0)
    tbase = sid * (ACC_ROWS // NS)
    for k in range(4):
        pltpu.async_copy(mb0, acc.at[pl.ds(tbase + k * CHUNK, CHUNK), :], sz)
    pltpu.async_copy(mb0.at[pl.ds(0, 120), :],
                     acc.at[pl.ds(tbase + 4 * CHUNK, 120), :], sz)
    fire(1, 1)
    fire(2, 2)
    for k in range(4):
        pltpu.make_async_copy(
            mb0, acc.at[pl.ds(tbase, CHUNK), :], sz).wait()
    pltpu.make_async_copy(
        mb0.at[pl.ds(0, 120), :], acc.at[pl.ds(tbase, 120), :], sz).wait()
    fire(0, 0)
    plsc.subcore_barrier()

    def group(g, carry):
        for b in range(3):
            j = g * 3 + b
            pltpu.make_async_copy(
                m0_hbm.at[pl.ds(0, CHUNK), :], mbs[b], sms[b]).wait()
            pltpu.make_async_copy(
                rows_hbm.at[wid, pl.ds(0, 1), :], ibs[b], sis[b]).wait()
            pltpu.sync_copy(mbs[b], acc.at[ibs[b].at[0]], add=True)

            @pl.when(j + 3 < NCH)
            def _refill():
                fire(b, j + 3)
        return carry

    lax.fori_loop(0, NCH // 3, group, 0)
    plsc.subcore_barrier()

    @pl.when(sid < NS - 1)
    def _copy_full():
        pltpu.sync_copy(acc.at[pl.ds(sid * 640, 640), :],
                        out_hbm.at[cid, pl.ds(sid * 640, 640), :])

    @pl.when(sid == NS - 1)
    def _copy_tail():
        pltpu.sync_copy(acc.at[pl.ds((NS - 1) * 640, N - (NS - 1) * 640), :],
                        out_hbm.at[cid, pl.ds((NS - 1) * 640, N - (NS - 1) * 640), :])


@functools.cache
def _get_scatter():
    return pl.kernel(
        _scatter_body,
        out_type=jax.ShapeDtypeStruct((NC, N, D), f32),
        mesh=plsc.VectorSubcoreMesh(core_axis_name="c", subcore_axis_name="s"),
        scratch_types=[
            pltpu.VMEM((CHUNK, D), f32), pltpu.VMEM((CHUNK, D), f32),
            pltpu.VMEM((CHUNK, D), f32),
            pltpu.VMEM((1, CHUNK), i32), pltpu.VMEM((1, CHUNK), i32),
            pltpu.VMEM((1, CHUNK), i32),
            pltpu.VMEM_SHARED((ACC_ROWS, D), f32),
            pltpu.SemaphoreType.DMA, pltpu.SemaphoreType.DMA,
            pltpu.SemaphoreType.DMA, pltpu.SemaphoreType.DMA,
            pltpu.SemaphoreType.DMA, pltpu.SemaphoreType.DMA,
            pltpu.SemaphoreType.DMA, pltpu.SemaphoreType.DMA,
            pltpu.SemaphoreType.DMA, pltpu.SemaphoreType.DMA,
        ],
    )


def _embed_tc(x_ref, ew_ref, eb_ref, w1a_ref, b1_ref, w1b_ref,
              h_ref, p_ref, q_ref):
    h = jnp.dot(x_ref[...], ew_ref[...], preferred_element_type=f32) + eb_ref[...]
    h_ref[...] = h
    p_ref[...] = jnp.dot(h, w1a_ref[...], preferred_element_type=f32) + b1_ref[...]
    q_ref[...] = jnp.dot(h, w1b_ref[...], preferred_element_type=f32)


BN = 2000
_full = lambda a, b: pl.BlockSpec((a, b), lambda j: (0, 0))
_blk = lambda: pl.BlockSpec((BN, D), lambda j: (j, 0))

_embed = pl.pallas_call(
    _embed_tc,
    grid=(N // BN,),
    in_specs=[_blk(), _full(D, D), _full(1, D), _full(D, D), _full(1, D),
              _full(D, D)],
    out_specs=[_blk(), _blk(), _blk()],
    out_shape=[jax.ShapeDtypeStruct((N, D), f32)] * 3,
)


def _edge_tc(tp_ref, tq_ref, attr_ref, w1c_ref, w2_ref, b2_ref, m_ref):
    u = tp_ref[...] + tq_ref[...] + attr_ref[...] * w1c_ref[...]
    u = u * jax.nn.sigmoid(u)
    v = jnp.dot(u.astype(bf16), w2_ref[...],
                preferred_element_type=f32) + b2_ref[...]
    m_ref[...] = v * jax.nn.sigmoid(v)


BE = 5184
_edge_mlp = pl.pallas_call(
    _edge_tc,
    grid=(E_HALF // BE,),
    in_specs=[pl.BlockSpec((BE, D), lambda j: (j, 0)),
              pl.BlockSpec((BE, D), lambda j: (j, 0)),
              pl.BlockSpec((BE, 1), lambda j: (j, 0)),
              _full(1, D), _full(D, D), _full(1, D)],
    out_specs=pl.BlockSpec((BE, D), lambda j: (j, 0)),
    out_shape=jax.ShapeDtypeStruct((E_HALF, D), f32),
)


def _node_tc(h_ref, a0_ref, a1_ref, nw1a_ref, nw1b_ref, nb1_ref, nw2_ref,
             nb2_ref, w1a_ref, b1_ref, w1b_ref, h_out, p_out, q_out):
    h = h_ref[...]
    agg = a0_ref[...] + a1_ref[...]
    t = (jnp.dot(h, nw1a_ref[...], preferred_element_type=f32)
         + jnp.dot(agg, nw1b_ref[...], preferred_element_type=f32)
         + nb1_ref[...])
    t = t * jax.nn.sigmoid(t)
    hn = jnp.dot(t, nw2_ref[...], preferred_element_type=f32) + nb2_ref[...]
    h_out[...] = hn
    p_out[...] = jnp.dot(hn, w1a_ref[...], preferred_element_type=f32) + b1_ref[...]
    q_out[...] = jnp.dot(hn, w1b_ref[...], preferred_element_type=f32)


_node = pl.pallas_call(
    _node_tc,
    grid=(N // BN,),
    in_specs=[_blk(), _blk(), _blk(),
              _full(D, D), _full(D, D), _full(1, D), _full(D, D), _full(1, D),
              _full(D, D), _full(1, D), _full(D, D)],
    out_specs=[_blk(), _blk(), _blk()],
    out_shape=[jax.ShapeDtypeStruct((N, D), f32)] * 3,
)


def _node_dec_tc(h_ref, a0_ref, a1_ref, nw1a_ref, nw1b_ref, nb1_ref,
                 nw2_ref, nb2_ref, dw1_ref, db1_ref, dw2_ref, db2_ref, o_ref):
    h = h_ref[...]
    agg = a0_ref[...] + a1_ref[...]
    t = (jnp.dot(h, nw1a_ref[...], preferred_element_type=f32)
         + jnp.dot(agg, nw1b_ref[...], preferred_element_type=f32)
         + nb1_ref[...])
    t = t * jax.nn.sigmoid(t)
    hn = jnp.dot(t, nw2_ref[...], preferred_element_type=f32) + nb2_ref[...]
    t2 = jnp.dot(hn, dw1_ref[...], preferred_element_type=f32) + db1_ref[...]
    t2 = t2 * jax.nn.sigmoid(t2)
    o_ref[...] = jnp.dot(t2, dw2_ref[...], preferred_element_type=f32) + db2_ref[...]


_node_dec = pl.pallas_call(
    _node_dec_tc,
    grid=(N // BN,),
    in_specs=[_blk(), _blk(), _blk(),
              _full(D, D), _full(D, D), _full(1, D), _full(D, D), _full(1, D),
              _full(D, D), _full(1, D), _full(D, 3), _full(1, 3)],
    out_specs=pl.BlockSpec((BN, 3), lambda j: (j, 0)),
    out_shape=jax.ShapeDtypeStruct((N, 3), f32),
)


def kernel(nodes, edges, edge_attr, emb_W, emb_b, edge_W1, edge_b1, edge_W2,
           edge_b2, node_W1, node_b1, node_W2, node_b2, dec_W1, dec_b1,
           dec_W2, dec_b2):
    row = edges[0]
    col = edges[1]
    padz = jnp.zeros((E_PAD - E,), i32)
    padn = N + jnp.arange(E_PAD - E, dtype=i32) % (ACC_ROWS - N)
    rowg = jnp.concatenate([row, padz]).reshape(2, NS, NCHT, CHUNK)
    colg = jnp.concatenate([col, padz]).reshape(2, NS, NCHT, CHUNK)
    rowsc = jnp.concatenate([row, padn]).reshape(NW, NCH, CHUNK)
    attr_p = jnp.concatenate(
        [edge_attr, jnp.zeros((E_PAD - E, 1), f32)], axis=0)
    attr_h = (attr_p[:E_HALF], attr_p[E_HALF:])

    h, p, q = _embed(nodes, emb_W, emb_b.reshape(1, D),
                     edge_W1[0, :D], edge_b1[0].reshape(1, D),
                     edge_W1[0, D:2 * D])
    out = None
    gather_fn = _get_gather()
    scatter_fn = _get_scatter()
    for i in range(NLAYERS):
        ms = []
        for half in range(2):
            tp, tq = gather_fn(p, q, rowg[half], colg[half])
            ms.append(_edge_mlp(tp, tq, attr_h[half],
                                edge_W1[i, 2 * D:].reshape(1, D),
                                edge_W2[i].astype(bf16),
                                edge_b2[i].reshape(1, D)))
        agg2 = scatter_fn(ms[0], ms[1], rowsc)
        nw1a = node_W1[i, :D]
        nw1b = node_W1[i, D:]
        if i + 1 < NLAYERS:
            h, p, q = _node(h, agg2[0], agg2[1], nw1a, nw1b,
                            node_b1[i].reshape(1, D), node_W2[i],
                            node_b2[i].reshape(1, D),
                            edge_W1[i + 1, :D], edge_b1[i + 1].reshape(1, D),
                            edge_W1[i + 1, D:2 * D])
        else:
            out = _node_dec(h, agg2[0], agg2[1], nw1a, nw1b,
                            node_b1[i].reshape(1, D), node_W2[i],
                            node_b2[i].reshape(1, D),
                            dec_W1, dec_b1.reshape(1, D),
                            dec_W2, dec_b2.reshape(1, 3))
    return out


# speedup vs baseline: 1.2428x; 1.0008x over previous
import functools

import jax
import jax.numpy as jnp
from jax import lax
from jax.experimental import pallas as pl
from jax.experimental.pallas import tpu as pltpu
from jax.experimental.pallas import tpu_sc as plsc

N = 10000
E = 320000
D = 128
NLAYERS = 4
NC, NS = 2, 16
NW = NC * NS
CHUNK = 128
NCH = 81
EW = NCH * CHUNK
E_PAD = NW * EW
E_HALF = E_PAD // 2
NCHT = E_HALF // (NS * CHUNK)
ACC_ROWS = 10112

f32 = jnp.float32
bf16 = jnp.bfloat16
i32 = jnp.int32


def _core_gather(sid, tbl_hbm, tbl_sh, idx_hbm, out_hbm, bufs, ibs,
                 sgs, sws, sis):
    base = sid * NCHT

    def fire_idx(b, j):
        pltpu.async_copy(idx_hbm.at[sid, pl.ds(j, 1), :], ibs[b], sis[b])

    def wait_idx(b):
        pltpu.make_async_copy(
            idx_hbm.at[sid, pl.ds(0, 1), :], ibs[b], sis[b]).wait()

    def fire_gather(b):
        pltpu.async_copy(tbl_sh.at[ibs[b].at[0]], bufs[b], sgs[b])

    def wait_gather(b):
        pltpu.make_async_copy(
            tbl_sh.at[ibs[b].at[0]], bufs[b], sgs[b]).wait()

    def wait_wb(b):
        pltpu.make_async_copy(
            bufs[b], out_hbm.at[pl.ds(0, CHUNK), :], sws[b]).wait()

    for b in range(3):
        fire_idx(b, b)

    @pl.when(sid < NS - 1)
    def _load_full():
        pltpu.sync_copy(tbl_hbm.at[pl.ds(sid * 640, 640), :],
                        tbl_sh.at[pl.ds(sid * 640, 640), :])

    @pl.when(sid == NS - 1)
    def _load_tail():
        pltpu.sync_copy(tbl_hbm.at[pl.ds((NS - 1) * 640, N - (NS - 1) * 640), :],
                        tbl_sh.at[pl.ds((NS - 1) * 640, N - (NS - 1) * 640), :])

    plsc.subcore_barrier()
    for b in range(2):
        wait_idx(b)
        fire_gather(b)

    def group(g, carry):
        for b in range(3):
            j = g * 3 + b
            b2 = (b + 2) % 3
            wait_gather(b)
            pltpu.async_copy(
                bufs[b], out_hbm.at[pl.ds((base + j) * CHUNK, CHUNK), :],
                sws[b])

            @pl.when(j >= 1)
            def _wait_prev_wb():
                wait_wb(b2)

            @pl.when(j + 2 < NCHT)
            def _fire_next_gather():
                wait_idx(b2)
                fire_gather(b2)

            @pl.when(j + 3 < NCHT)
            def _fire_next_idx():
                fire_idx(b, j + 3)
        return carry

    lax.fori_loop(0, NCHT // 3, group, 0)
    wait_wb((NCHT - 1) % 3)


def _gather_body(p_hbm, q_hbm, rowg_hbm, colg_hbm, tp_hbm, tq_hbm,
                 b0, b1, b2, ib0, ib1, ib2, tbl_sh,
                 sg0, sg1, sg2, sw0, sw1, sw2, si0, si1, si2):
    cid = lax.axis_index("c")
    sid = lax.axis_index("s")
    bufs = (b0, b1, b2)
    ibs = (ib0, ib1, ib2)
    sgs = (sg0, sg1, sg2)
    sws = (sw0, sw1, sw2)
    sis = (si0, si1, si2)

    @pl.when(cid == 0)
    def _core0():
        _core_gather(sid, p_hbm, tbl_sh, rowg_hbm, tp_hbm, bufs, ibs,
                     sgs, sws, sis)

    @pl.when(cid == 1)
    def _core1():
        _core_gather(sid, q_hbm, tbl_sh, colg_hbm, tq_hbm, bufs, ibs,
                     sgs, sws, sis)


@functools.cache
def _get_gather():
    return pl.kernel(
        _gather_body,
        out_type=[jax.ShapeDtypeStruct((E_HALF, D), f32),
                  jax.ShapeDtypeStruct((E_HALF, D), f32)],
        mesh=plsc.VectorSubcoreMesh(core_axis_name="c", subcore_axis_name="s"),
        scratch_types=(
            [pltpu.VMEM((CHUNK, D), f32)] * 3
            + [pltpu.VMEM((1, CHUNK), i32)] * 3
            + [pltpu.VMEM_SHARED((N, D), f32)]
            + [pltpu.SemaphoreType.DMA] * 9
        ),
    )


def _scatter_body(m0_hbm, m1_hbm, rows_hbm, out_hbm, mb0, mb1, mb2,
                  ib0, ib1, ib2, acc, sm0, sm1, sm2, si0, si1, si2,
                  sz, sa0, sa1, sa2):
    cid = lax.axis_index("c")
    sid = lax.axis_index("s")
    wid = sid * NC + cid
    mbs = (mb0, mb1, mb2)
    ibs = (ib0, ib1, ib2)
    sms = (sm0, sm1, sm2)
    sis = (si0, si1, si2)
    sas = (sa0, sa1, sa2)
    base = wid * NCH

    def fire(b, j):
        @pl.when(wid < NW // 2)
        def _fire0():
            pltpu.async_copy(
                m0_hbm.at[pl.ds((base + j) * CHUNK, CHUNK), :], mbs[b], sms[b])

        @pl.when(wid >= NW // 2)
        def _fire1():
            pltpu.async_copy(
                m1_hbm.at[pl.ds((base - NW // 2 * NCH + j) * CHUNK, CHUNK), :],
                mbs[b], sms[b])

        pltpu.async_copy(rows_hbm.at[wid, pl.ds(j, 1), :], ibs[b], sis[b])

    def zrow(r, carry):
        for c8 in range(D // 16):
            mb0[r, pl.ds(c8 * 16, 16)] = jnp.zeros((16,), f32)
        return carry

    lax.fori_loop(0, CHUNK, zrow, 0)
    tbase = sid * (ACC_ROWS // NS)
    for k in range(4):
        pltpu.async_copy(mb0, acc.at[pl.ds(tbase + k * CHUNK, CHUNK), :], sz)
    pltpu.async_copy(mb0.at[pl.ds(0, 120), :],
                     acc.at[pl.ds(tbase + 4 * CHUNK, 120), :], sz)
    fire(1, 1)
    fire(2, 2)
    for k in range(4):
        pltpu.make_async_copy(
            mb0, acc.at[pl.ds(tbase, CHUNK), :], sz).wait()
    pltpu.make_async_copy(
        mb0.at[pl.ds(0, 120), :], acc.at[pl.ds(tbase, 120), :], sz).wait()
    fire(0, 0)
    plsc.subcore_barrier()

    def group(g, carry):
        for b in range(3):
            j = g * 3 + b
            pltpu.make_async_copy(
                m0_hbm.at[pl.ds(0, CHUNK), :], mbs[b], sms[b]).wait()
            pltpu.make_async_copy(
                rows_hbm.at[wid, pl.ds(0, 1), :], ibs[b], sis[b]).wait()
            pltpu.sync_copy(mbs[b], acc.at[ibs[b].at[0]], add=True)

            @pl.when(j + 3 < NCH)
            def _refill():
                fire(b, j + 3)
        return carry

    lax.fori_loop(0, NCH // 3, group, 0)
    plsc.subcore_barrier()

    @pl.when(sid < NS - 1)
    def _copy_full():
        pltpu.sync_copy(acc.at[pl.ds(sid * 640, 640), :],
                        out_hbm.at[cid, pl.ds(sid * 640, 640), :])

    @pl.when(sid == NS - 1)
    def _copy_tail():
        pltpu.sync_copy(acc.at[pl.ds((NS - 1) * 640, N - (NS - 1) * 640), :],
                        out_hbm.at[cid, pl.ds((NS - 1) * 640, N - (NS - 1) * 640), :])


@functools.cache
def _get_scatter():
    return pl.kernel(
        _scatter_body,
        out_type=jax.ShapeDtypeStruct((NC, N, D), f32),
        mesh=plsc.VectorSubcoreMesh(core_axis_name="c", subcore_axis_name="s"),
        scratch_types=[
            pltpu.VMEM((CHUNK, D), f32), pltpu.VMEM((CHUNK, D), f32),
            pltpu.VMEM((CHUNK, D), f32),
            pltpu.VMEM((1, CHUNK), i32), pltpu.VMEM((1, CHUNK), i32),
            pltpu.VMEM((1, CHUNK), i32),
            pltpu.VMEM_SHARED((ACC_ROWS, D), f32),
            pltpu.SemaphoreType.DMA, pltpu.SemaphoreType.DMA,
            pltpu.SemaphoreType.DMA, pltpu.SemaphoreType.DMA,
            pltpu.SemaphoreType.DMA, pltpu.SemaphoreType.DMA,
            pltpu.SemaphoreType.DMA, pltpu.SemaphoreType.DMA,
            pltpu.SemaphoreType.DMA, pltpu.SemaphoreType.DMA,
        ],
    )


def _embed_tc(x_ref, ew_ref, eb_ref, w1a_ref, b1_ref, w1b_ref,
              h_ref, p_ref, q_ref):
    h = jnp.dot(x_ref[...], ew_ref[...], preferred_element_type=f32) + eb_ref[...]
    h_ref[...] = h
    p_ref[...] = jnp.dot(h, w1a_ref[...], preferred_element_type=f32) + b1_ref[...]
    q_ref[...] = jnp.dot(h, w1b_ref[...], preferred_element_type=f32)


BN = 10000
_full = lambda a, b: pl.BlockSpec((a, b), lambda j: (0, 0))
_blk = lambda: pl.BlockSpec((BN, D), lambda j: (j, 0))

_embed = pl.pallas_call(
    _embed_tc,
    grid=(N // BN,),
    in_specs=[_blk(), _full(D, D), _full(1, D), _full(D, D), _full(1, D),
              _full(D, D)],
    out_specs=[_blk(), _blk(), _blk()],
    out_shape=[jax.ShapeDtypeStruct((N, D), f32)] * 3,
)


def _edge_tc(tp_ref, tq_ref, attr_ref, w1c_ref, w2_ref, b2_ref, m_ref):
    u = tp_ref[...] + tq_ref[...] + attr_ref[...] * w1c_ref[...]
    u = u * jax.nn.sigmoid(u)
    v = jnp.dot(u.astype(bf16), w2_ref[...],
                preferred_element_type=f32) + b2_ref[...]
    m_ref[...] = v * jax.nn.sigmoid(v)


BE = 5184
_edge_mlp = pl.pallas_call(
    _edge_tc,
    grid=(E_HALF // BE,),
    in_specs=[pl.BlockSpec((BE, D), lambda j: (j, 0)),
              pl.BlockSpec((BE, D), lambda j: (j, 0)),
              pl.BlockSpec((BE, 1), lambda j: (j, 0)),
              _full(1, D), _full(D, D), _full(1, D)],
    out_specs=pl.BlockSpec((BE, D), lambda j: (j, 0)),
    out_shape=jax.ShapeDtypeStruct((E_HALF, D), f32),
)


def _node_tc(h_ref, a0_ref, a1_ref, nw1a_ref, nw1b_ref, nb1_ref, nw2_ref,
             nb2_ref, w1a_ref, b1_ref, w1b_ref, h_out, p_out, q_out):
    h = h_ref[...]
    agg = a0_ref[...] + a1_ref[...]
    t = (jnp.dot(h, nw1a_ref[...], preferred_element_type=f32)
         + jnp.dot(agg, nw1b_ref[...], preferred_element_type=f32)
         + nb1_ref[...])
    t = t * jax.nn.sigmoid(t)
    hn = jnp.dot(t, nw2_ref[...], preferred_element_type=f32) + nb2_ref[...]
    h_out[...] = hn
    p_out[...] = jnp.dot(hn, w1a_ref[...], preferred_element_type=f32) + b1_ref[...]
    q_out[...] = jnp.dot(hn, w1b_ref[...], preferred_element_type=f32)


_node = pl.pallas_call(
    _node_tc,
    grid=(N // BN,),
    in_specs=[_blk(), _blk(), _blk(),
              _full(D, D), _full(D, D), _full(1, D), _full(D, D), _full(1, D),
              _full(D, D), _full(1, D), _full(D, D)],
    out_specs=[_blk(), _blk(), _blk()],
    out_shape=[jax.ShapeDtypeStruct((N, D), f32)] * 3,
)


def _node_dec_tc(h_ref, a0_ref, a1_ref, nw1a_ref, nw1b_ref, nb1_ref,
                 nw2_ref, nb2_ref, dw1_ref, db1_ref, dw2_ref, db2_ref, o_ref):
    h = h_ref[...]
    agg = a0_ref[...] + a1_ref[...]
    t = (jnp.dot(h, nw1a_ref[...], preferred_element_type=f32)
         + jnp.dot(agg, nw1b_ref[...], preferred_element_type=f32)
         + nb1_ref[...])
    t = t * jax.nn.sigmoid(t)
    hn = jnp.dot(t, nw2_ref[...], preferred_element_type=f32) + nb2_ref[...]
    t2 = jnp.dot(hn, dw1_ref[...], preferred_element_type=f32) + db1_ref[...]
    t2 = t2 * jax.nn.sigmoid(t2)
    o_ref[...] = jnp.dot(t2, dw2_ref[...], preferred_element_type=f32) + db2_ref[...]


_node_dec = pl.pallas_call(
    _node_dec_tc,
    grid=(N // BN,),
    in_specs=[_blk(), _blk(), _blk(),
              _full(D, D), _full(D, D), _full(1, D), _full(D, D), _full(1, D),
              _full(D, D), _full(1, D), _full(D, 3), _full(1, 3)],
    out_specs=pl.BlockSpec((BN, 3), lambda j: (j, 0)),
    out_shape=jax.ShapeDtypeStruct((N, 3), f32),
)


def kernel(nodes, edges, edge_attr, emb_W, emb_b, edge_W1, edge_b1, edge_W2,
           edge_b2, node_W1, node_b1, node_W2, node_b2, dec_W1, dec_b1,
           dec_W2, dec_b2):
    row = edges[0]
    col = edges[1]
    padz = jnp.zeros((E_PAD - E,), i32)
    padn = N + jnp.arange(E_PAD - E, dtype=i32) % (ACC_ROWS - N)
    rowg = jnp.concatenate([row, padz]).reshape(2, NS, NCHT, CHUNK)
    colg = jnp.concatenate([col, padz]).reshape(2, NS, NCHT, CHUNK)
    rowsc = jnp.concatenate([row, padn]).reshape(NW, NCH, CHUNK)
    attr_p = jnp.concatenate(
        [edge_attr, jnp.zeros((E_PAD - E, 1), f32)], axis=0)
    attr_h = (attr_p[:E_HALF], attr_p[E_HALF:])

    h, p, q = _embed(nodes, emb_W, emb_b.reshape(1, D),
                     edge_W1[0, :D], edge_b1[0].reshape(1, D),
                     edge_W1[0, D:2 * D])
    out = None
    gather_fn = _get_gather()
    scatter_fn = _get_scatter()
    for i in range(NLAYERS):
        ms = []
        for half in range(2):
            tp, tq = gather_fn(p, q, rowg[half], colg[half])
            ms.append(_edge_mlp(tp, tq, attr_h[half],
                                edge_W1[i, 2 * D:].reshape(1, D),
                                edge_W2[i].astype(bf16),
                                edge_b2[i].reshape(1, D)))
        agg2 = scatter_fn(ms[0], ms[1], rowsc)
        nw1a = node_W1[i, :D]
        nw1b = node_W1[i, D:]
        if i + 1 < NLAYERS:
            h, p, q = _node(h, agg2[0], agg2[1], nw1a, nw1b,
                            node_b1[i].reshape(1, D), node_W2[i],
                            node_b2[i].reshape(1, D),
                            edge_W1[i + 1, :D], edge_b1[i + 1].reshape(1, D),
                            edge_W1[i + 1, D:2 * D])
        else:
            out = _node_dec(h, agg2[0], agg2[1], nw1a, nw1b,
                            node_b1[i].reshape(1, D), node_W2[i],
                            node_b2[i].reshape(1, D),
                            dec_W1, dec_b1.reshape(1, D),
                            dec_W2, dec_b2.reshape(1, 3))
    return out
